# baseline (speedup 1.0000x reference)
import functools

import jax
import jax.numpy as jnp
from jax import lax
from jax.experimental import pallas as pl
from jax.experimental.pallas import tpu as pltpu
from jax.experimental.pallas import tpu_sc as plsc

D = 256
NC, NS = 2, 16
NW = NC * NS
N_GRID, N_LAT, D_IN = 10000, 2562, 78
EP_P = 16384
EP_E = 30720
LAT_P = 2688
GRID_P = 10240
T_EDGE = 256
T_LAT = 128
T_GRID = 400
F32 = jnp.float32


def _vecs(*vs):
    w = max(v.shape[0] for v in vs)
    out = jnp.zeros((8, w), F32)
    for i, v in enumerate(vs):
        out = out.at[i, : v.shape[0]].set(v)
    return out


def _ln(h, g, b):
    bf = jnp.bfloat16
    hb = h.astype(bf)
    ones = jnp.full((h.shape[-1], 8), 1.0 / h.shape[-1], bf)
    mu = _dot(hb, ones)[:, :1]
    mu2 = _dot((hb * hb).astype(bf), ones)[:, :1]
    var = jnp.maximum(mu2 - mu * mu, 0.0)
    return (h - mu) * lax.rsqrt(var + 1e-5) * g + b


def _dot(a, b):
    return jnp.dot(a, b, preferred_element_type=F32)


def _pack_bf16(x):
    lo = lax.bitcast_convert_type(x[:, :D // 2].astype(jnp.bfloat16),
                                  jnp.uint16).astype(jnp.uint32)
    hi = lax.bitcast_convert_type(x[:, D // 2:].astype(jnp.bfloat16),
                                  jnp.uint16).astype(jnp.uint32)
    return lax.bitcast_convert_type(lo | (hi << 16), jnp.int32)


def _unpack_bf16(xi):
    xu = lax.bitcast_convert_type(xi, jnp.uint32)
    lo = lax.bitcast_convert_type((xu & 0xFFFF).astype(jnp.uint16), jnp.bfloat16)
    hi = lax.bitcast_convert_type((xu >> 16).astype(jnp.uint16), jnp.bfloat16)
    return lo, hi


_CP = pltpu.CompilerParams(dimension_semantics=("parallel",))


def _mlp_ln(x, w1, vecs, w2, T, want_packed=False):
    n, din = x.shape
    dh, do = w1.shape[1], w2.shape[1]

    def body(x_r, w1_r, w2_r, v_r, o_r, *po):
        bf = jnp.bfloat16
        h = jnp.maximum(
            _dot(x_r[...].astype(bf), w1_r[...]) + v_r[0:1, :dh], 0.0)
        h = _dot(h.astype(bf), w2_r[...]) + v_r[1:2, :do]
        y = _ln(h, v_r[2:3, :do], v_r[3:4, :do])
        o_r[...] = y
        if want_packed:
            po[0][...] = _pack_bf16(y)

    out_specs = [pl.BlockSpec((T, do), lambda i: (i, 0))]
    out_shape = [jax.ShapeDtypeStruct((n, do), F32)]
    if want_packed:
        out_specs.append(pl.BlockSpec((T, do // 2), lambda i: (i, 0)))
        out_shape.append(jax.ShapeDtypeStruct((n, do // 2), jnp.int32))
    r = pl.pallas_call(
        body,
        grid=(n // T,),
        in_specs=[
            pl.BlockSpec((T, din), lambda i: (i, 0)),
            pl.BlockSpec((din, dh), lambda i: (0, 0)),
            pl.BlockSpec((dh, do), lambda i: (0, 0)),
            pl.BlockSpec((8, vecs.shape[1]), lambda i: (0, 0)),
        ],
        out_specs=out_specs if want_packed else out_specs[0],
        out_shape=out_shape if want_packed else out_shape[0],
        compiler_params=_CP,
    )(x, w1, w2, vecs)
    return r


def _edge_mlp(xs_arr, xs_off, xd_arr, xd_off, e, w1s, w1d, w1e, vecs, w2,
              n_edges, want_enew):
    nb = n_edges // T_EDGE
    has_xd = xd_arr is not None

    def body(*refs):
        if has_xd:
            xs_r, xd_r, e_r, w1s_r, w1d_r, w1e_r, w2_r, v_r, *outs = refs
        else:
            xs_r, e_r, w1s_r, w1e_r, w2_r, v_r, *outs = refs
        bf = jnp.bfloat16
        hd = D // 2
        lo, hi = _unpack_bf16(xs_r[...])
        acc = _dot(lo, w1s_r[:hd]) + _dot(hi, w1s_r[hd:])
        if has_xd:
            lo, hi = _unpack_bf16(xd_r[...])
            acc = acc + _dot(lo, w1d_r[:hd]) + _dot(hi, w1d_r[hd:])
        acc = acc + _dot(e_r[...].astype(bf), w1e_r[...]) + v_r[0:1, :]
        h = jnp.maximum(acc, 0.0)
        h = _dot(h.astype(bf), w2_r[...]) + v_r[1:2, :]
        m = _ln(h, v_r[2:3, :], v_r[3:4, :])
        outs[0][...] = m
        if want_enew:
            outs[1][...] = e_r[...] + m

    in_specs = [pl.BlockSpec((T_EDGE, D // 2), lambda i, o=xs_off: (i + o, 0))]
    inputs = [xs_arr]
    if has_xd:
        in_specs.append(
            pl.BlockSpec((T_EDGE, D // 2), lambda i, o=xd_off: (i + o, 0)))
        inputs.append(xd_arr)
    in_specs.append(pl.BlockSpec((T_EDGE, D), lambda i: (i, 0)))
    inputs.append(e)
    wspec = pl.BlockSpec((D, D), lambda i: (0, 0))
    in_specs += [wspec, wspec] if not has_xd else [wspec, wspec, wspec]
    inputs += [w1s, w1e] if not has_xd else [w1s, w1d, w1e]
    in_specs += [wspec, pl.BlockSpec((8, D), lambda i: (0, 0))]
    inputs += [w2, vecs]

    out_spec = pl.BlockSpec((T_EDGE, D), lambda i: (i, 0))
    out_shape = jax.ShapeDtypeStruct((n_edges, D), F32)
    if want_enew:
        out_specs, out_shapes = [out_spec, out_spec], [out_shape, out_shape]
    else:
        out_specs, out_shapes = out_spec, out_shape

    return pl.pallas_call(
        body,
        grid=(nb,),
        in_specs=in_specs,
        out_specs=out_specs,
        out_shape=out_shapes,
        compiler_params=_CP,
    )(*inputs)


def _node_mlp(x, aggs, w1x, w1a, vecs, w2, n_rows, T, want_packed=False):
    use_x = x is not None

    def body(*refs):
        refs = list(refs)
        p_r = refs.pop() if want_packed else None
        o_r = refs.pop()
        v_r = refs.pop()
        w2_r = refs.pop()
        x_r = refs.pop(0) if use_x else None
        agg_rs = [refs.pop(0) for _ in aggs]
        w1x_r = refs.pop(0) if use_x else None
        w1a_r = refs.pop(0) if aggs else None
        acc = v_r[0:1, :] * jnp.ones((T, 1), F32)
        if use_x:
            acc = acc + _dot(x_r[...], w1x_r[...])
        if agg_rs:
            a = agg_rs[0][...]
            for r in agg_rs[1:]:
                a = a + r[...]
            acc = acc + _dot(a, w1a_r[...])
        h = jnp.maximum(acc, 0.0)
        h = _dot(h, w2_r[...]) + v_r[1:2, :]
        m = _ln(h, v_r[2:3, :], v_r[3:4, :])
        y = (x_r[...] + m) if use_x else m
        o_r[...] = y
        if want_packed:
            p_r[...] = _pack_bf16(y)

    rspec = pl.BlockSpec((T, D), lambda i: (i, 0))
    wspec = pl.BlockSpec((D, D), lambda i: (0, 0))
    inputs, in_specs = [], []
    if use_x:
        inputs.append(x); in_specs.append(rspec)
    for a in aggs:
        inputs.append(a); in_specs.append(rspec)
    if use_x:
        inputs.append(w1x); in_specs.append(wspec)
    if aggs:
        inputs.append(w1a); in_specs.append(wspec)
    inputs += [w2, vecs]
    in_specs += [wspec, pl.BlockSpec((8, D), lambda i: (0, 0))]

    out_specs = [rspec]
    out_shape = [jax.ShapeDtypeStruct((n_rows, D), F32)]
    if want_packed:
        out_specs.append(pl.BlockSpec((T, D // 2), lambda i: (i, 0)))
        out_shape.append(jax.ShapeDtypeStruct((n_rows, D // 2), jnp.int32))
    return pl.pallas_call(
        body,
        grid=(n_rows // T,),
        in_specs=in_specs,
        out_specs=out_specs if want_packed else out_specs[0],
        out_shape=out_shape if want_packed else out_shape[0],
        compiler_params=_CP,
    )(*inputs)


def _final_mlp(x, featp, w1, vecs, w2):
    n = x.shape[0]
    dh = w1.shape[1]

    def body(x_r, f_r, w1_r, w2_r, v_r, o_r):
        h = jnp.maximum(_dot(x_r[...], w1_r[...]) + v_r[0:1, :], 0.0)
        o_r[...] = _dot(h, w2_r[...]) + v_r[1:2, :] + f_r[...]

    return pl.pallas_call(
        body,
        grid=(n // T_GRID,),
        in_specs=[
            pl.BlockSpec((T_GRID, D), lambda i: (i, 0)),
            pl.BlockSpec((T_GRID, dh), lambda i: (i, 0)),
            pl.BlockSpec((D, dh), lambda i: (0, 0)),
            pl.BlockSpec((dh, dh), lambda i: (0, 0)),
            pl.BlockSpec((8, dh), lambda i: (0, 0)),
        ],
        out_specs=pl.BlockSpec((T_GRID, dh), lambda i: (i, 0)),
        out_shape=jax.ShapeDtypeStruct((n, dh), F32),
        compiler_params=_CP,
    )(x, featp, w1, w2, vecs)


@functools.cache
def _mesh():
    return plsc.VectorSubcoreMesh(core_axis_name="c", subcore_axis_name="s",
                                  num_cores=NC, num_subcores=NS)


_CHUNK = 128


def _sc_gather(table, idx, n_out):
    return _sc_gather_raw(table, idx, n_out, table.shape[1])


def _sc_gather_raw(table, idx, n_out, w):
    bpw = n_out // NW
    nfull, rem = divmod(bpw, _CHUNK)
    cps = [_CHUNK] * nfull + ([rem] if rem else [])
    nch = len(cps)
    dt = table.dtype
    scratch = [pltpu.VMEM((bpw,), jnp.int32),
               pltpu.VMEM((_CHUNK, w), dt), pltpu.VMEM((_CHUNK, w), dt),
               pltpu.SemaphoreType.DMA, pltpu.SemaphoreType.DMA,
               pltpu.SemaphoreType.DMA, pltpu.SemaphoreType.DMA]

    @functools.partial(
        pl.kernel,
        out_type=jax.ShapeDtypeStruct((n_out, w), dt),
        mesh=_mesh(),
        scratch_types=scratch,
    )
    def k(table_hbm, idx_hbm, out_hbm, idx_v, r0, r1, sg0, sg1, sw0, sw1):
        wid = lax.axis_index("s") * NC + lax.axis_index("c")
        base = wid * bpw
        pltpu.sync_copy(idx_hbm.at[pl.ds(base, bpw)], idx_v)
        rows, sgs, sws = (r0, r1), (sg0, sg1), (sw0, sw1)

        def buf(i):
            b = i & 1
            sz = cps[i]
            return rows[b] if sz == _CHUNK else rows[b].at[pl.ds(0, sz)]

        def issue_g(i):
            sz = cps[i]
            return pltpu.async_copy(
                table_hbm.at[idx_v.at[pl.ds(i * _CHUNK, sz)]], buf(i),
                sgs[i & 1])

        def issue_w(i):
            sz = cps[i]
            return pltpu.async_copy(
                buf(i), out_hbm.at[pl.ds(base + i * _CHUNK, sz)], sws[i & 1])

        gd, wd = [None] * nch, [None] * nch
        gd[0] = issue_g(0)
        for i in range(nch):
            if i + 1 < nch:
                if i >= 1:
                    wd[i - 1].wait()
                gd[i + 1] = issue_g(i + 1)
            gd[i].wait()
            wd[i] = issue_w(i)
        wd[nch - 1].wait()
        if nch >= 2:
            wd[nch - 2].wait()

    return k(table, idx)


def _sc_scatter(m, dst, zeros, n_edges, n_nodes):
    dh = D // NC
    epw = n_edges // NS
    nch = epw // _CHUNK
    assert nch * _CHUNK == epw
    rps = n_nodes // NS
    scratch = [pltpu.VMEM((1, _CHUNK), jnp.int32), pltpu.VMEM((1, _CHUNK), jnp.int32),
               pltpu.VMEM((_CHUNK, dh), F32), pltpu.VMEM((_CHUNK, dh), F32),
               pltpu.SemaphoreType.DMA, pltpu.SemaphoreType.DMA,
               pltpu.SemaphoreType.DMA, pltpu.SemaphoreType.DMA,
               pltpu.SemaphoreType.DMA, pltpu.SemaphoreType.DMA,
               pltpu.VMEM_SHARED((n_nodes, dh), F32)]

    @functools.partial(
        pl.kernel,
        out_type=jax.ShapeDtypeStruct((n_nodes, D), F32),
        mesh=_mesh(),
        scratch_types=scratch,
    )
    def k(m_hbm, dst_hbm, z_hbm, out_hbm, i0, i1, m0, m1,
          si0, si1, sm0, sm1, ss0, ss1, acc):
        cid = lax.axis_index("c")
        sid = lax.axis_index("s")
        col = cid * dh
        pltpu.sync_copy(z_hbm, acc.at[pl.ds(sid * rps, rps)])
        plsc.subcore_barrier()
        base = sid * epw
        idxs, mbufs = (i0, i1), (m0, m1)
        sis, sms, sss = (si0, si1), (sm0, sm1), (ss0, ss1)

        def issue_l(i):
            b = i & 1
            off = base + i * _CHUNK
            di = pltpu.async_copy(dst_hbm.at[pl.ds(off, _CHUNK)],
                                  idxs[b].at[0], sis[b])
            dm = pltpu.async_copy(m_hbm.at[pl.ds(off, _CHUNK), pl.ds(col, dh)],
                                  mbufs[b], sms[b])
            return di, dm

        def issue_s(i):
            b = i & 1
            return pltpu.async_copy(mbufs[b], acc.at[idxs[b].at[0]], sss[b],
                                    add=True)

        ld, sd = [None] * nch, [None] * nch
        ld[0] = issue_l(0)
        for i in range(nch):
            if i + 1 < nch:
                if i >= 1:
                    sd[i - 1].wait()
                ld[i + 1] = issue_l(i + 1)
            ld[i][0].wait()
            ld[i][1].wait()
            sd[i] = issue_s(i)
        sd[nch - 1].wait()
        if nch >= 2:
            sd[nch - 2].wait()
        plsc.subcore_barrier()
        pltpu.sync_copy(acc.at[pl.ds(sid * rps, rps)],
                        out_hbm.at[pl.ds(sid * rps, rps), pl.ds(col, dh)])

    return k(m, dst, zeros)


def _pad_idx(a, n, fill_start, fill_mod):
    npad = n - a.shape[0]
    pad = fill_start + (jnp.arange(npad, dtype=jnp.int32) % fill_mod)
    return jnp.concatenate([a.astype(jnp.int32), pad])


def _pad2(a, rows, cols):
    return jnp.pad(a, ((0, rows - a.shape[0]), (0, cols - a.shape[1])))


def kernel(features, params, graph):
    f = features[0]
    P, G = params, graph

    fpad = _pad2(f, N_GRID, 128)
    zeros_lat = jnp.zeros((LAT_P // NS, D // NC), F32)
    zeros_grid = jnp.zeros((GRID_P // NS, D // NC), F32)

    enc_src = _pad_idx(G["enc_src"], EP_E, 0, N_GRID)
    enc_dst = _pad_idx(G["enc_dst"], EP_E, N_LAT, LAT_P - N_LAT)
    proc_srcdst = jnp.concatenate([
        _pad_idx(G["proc_src"], EP_P, 0, N_LAT),
        _pad_idx(G["proc_dst"], EP_P, 0, N_LAT)])
    proc_dst = _pad_idx(G["proc_dst"], EP_P, N_LAT, LAT_P - N_LAT)
    dec_srcdst = jnp.concatenate([
        _pad_idx(G["dec_src"], EP_E, 0, N_LAT),
        LAT_P + _pad_idx(G["dec_dst"], EP_E, 0, N_GRID)])
    dec_dst_s = _pad_idx(G["dec_dst"], EP_E, N_GRID, GRID_P - N_GRID)

    bfc = lambda a: a.astype(jnp.bfloat16)
    pne = P["node_encoder"]
    x_grid, x_grid_pk = _mlp_ln(fpad, bfc(_pad2(pne["w1"], 128, D)),
                                _vecs(pne["b1"], pne["b2"], pne["ln_g"],
                                      pne["ln_b"]),
                                bfc(pne["w2"]), T_GRID, want_packed=True)

    def enc_attr(p, attr, n_pad):
        return _mlp_ln(_pad2(attr, n_pad, 8), bfc(_pad2(p["w1"], 8, D)),
                       _vecs(p["b1"], p["b2"], p["ln_g"], p["ln_b"]),
                       bfc(p["w2"]), T_EDGE)

    e_enc = enc_attr(P["enc_edge_encoder"], G["enc_attr"], EP_E)
    ep = enc_attr(P["proc_edge_encoder"], G["proc_attr"], EP_P)
    ed = enc_attr(P["dec_edge_encoder"], G["dec_attr"], EP_E)

    def edge_w(p):
        w1 = p["w1"].astype(jnp.bfloat16)
        return (w1[:D], w1[D:2 * D], w1[2 * D:],
                _vecs(p["b1"], p["b2"], p["ln_g"], p["ln_b"]),
                p["w2"].astype(jnp.bfloat16))

    def node_w(p):
        w1 = p["w1"]
        return (w1[:D], w1[D:],
                _vecs(p["b1"], p["b2"], p["ln_g"], p["ln_b"]), p["w2"])

    w1s, _, w1e, vecs, w2 = edge_w(P["enc_block"]["edge"])
    xs = _sc_gather(x_grid_pk, enc_src, EP_E)
    m = _edge_mlp(xs, 0, None, 0, e_enc, w1s, None, w1e, vecs, w2, EP_E, False)
    agg = _sc_scatter(m, enc_dst, zeros_lat, EP_E, LAT_P)
    w1x, w1a, nvecs, nw2 = node_w(P["enc_block"]["node"])
    x_lat, x_lat_pk = _node_mlp(None, [agg], None, w1a, nvecs, nw2, LAT_P,
                                T_LAT, want_packed=True)
    x_grid, x_grid_pk = _node_mlp(x_grid, [], w1x, None, nvecs, nw2, N_GRID,
                                  T_GRID, want_packed=True)

    nb_p = EP_P // T_EDGE
    for bp in P["proc_blocks"]:
        w1s, w1d, w1e, vecs, w2 = edge_w(bp["edge"])
        rows = _sc_gather(x_lat_pk, proc_srcdst, 2 * EP_P)
        m, ep = _edge_mlp(rows, 0, rows, nb_p, ep, w1s, w1d, w1e, vecs, w2,
                          EP_P, True)
        agg = _sc_scatter(m, proc_dst, zeros_lat, EP_P, LAT_P)
        w1x, w1a, nvecs, nw2 = node_w(bp["node"])
        x_lat, x_lat_pk = _node_mlp(x_lat, [agg], w1x, w1a, nvecs, nw2, LAT_P,
                                    T_LAT, want_packed=True)

    w1s, w1d, w1e, vecs, w2 = edge_w(P["dec_block"]["edge"])
    table = jnp.concatenate([x_lat_pk, x_grid_pk])
    rows = _sc_gather(table, dec_srcdst, 2 * EP_E)
    nb_e = EP_E // T_EDGE
    m = _edge_mlp(rows, 0, rows, nb_e, ed, w1s, w1d, w1e, vecs, w2, EP_E, False)
    agg = _sc_scatter(m, dec_dst_s, zeros_grid, EP_E, GRID_P)
    w1x, w1a, nvecs, nw2 = node_w(P["dec_block"]["node"])
    x_grid = _node_mlp(x_grid, [agg], w1x, w1a, nvecs, nw2, N_GRID, T_GRID)

    pd = P["node_decoder"]
    out = _final_mlp(x_grid, fpad, pd["w1"],
                     _vecs(pd["b1"], jnp.pad(pd["b2"], (0, 128 - D_IN))),
                     _pad2(pd["w2"], 128, 128))
    return out[:, :D_IN][None]


# speedup vs baseline: 1.0420x; 1.0420x over previous
import functools

import jax
import jax.numpy as jnp
from jax import lax
from jax.experimental import pallas as pl
from jax.experimental.pallas import tpu as pltpu
from jax.experimental.pallas import tpu_sc as plsc

D = 256
NC, NS = 2, 16
NW = NC * NS
N_GRID, N_LAT, D_IN = 10000, 2562, 78
EP_P = 16384
EP_E = 30720
LAT_P = 2688
GRID_P = 10240
T_EDGE = 256
T_LAT = 128
T_GRID = 400
F32 = jnp.float32


def _vecs(*vs):
    w = max(v.shape[0] for v in vs)
    out = jnp.zeros((8, w), F32)
    for i, v in enumerate(vs):
        out = out.at[i, : v.shape[0]].set(v)
    return out


def _ln(h, g, b):
    mu = jnp.mean(h, axis=-1, keepdims=True)
    var = jnp.mean((h - mu) ** 2, axis=-1, keepdims=True)
    return (h - mu) * lax.rsqrt(var + 1e-5) * g + b


def _dot(a, b):
    return jnp.dot(a, b, preferred_element_type=F32)


def _pack_bf16(x):
    lo = lax.bitcast_convert_type(x[:, :D // 2].astype(jnp.bfloat16),
                                  jnp.uint16).astype(jnp.uint32)
    hi = lax.bitcast_convert_type(x[:, D // 2:].astype(jnp.bfloat16),
                                  jnp.uint16).astype(jnp.uint32)
    return lax.bitcast_convert_type(lo | (hi << 16), jnp.int32)


def _unpack_bf16(xi):
    xu = lax.bitcast_convert_type(xi, jnp.uint32)
    lo = lax.bitcast_convert_type((xu & 0xFFFF).astype(jnp.uint16), jnp.bfloat16)
    hi = lax.bitcast_convert_type((xu >> 16).astype(jnp.uint16), jnp.bfloat16)
    return lo, hi


_CP = pltpu.CompilerParams(dimension_semantics=("parallel",))


def _mlp_ln(x, w1, vecs, w2, T, want_packed=False):
    n, din = x.shape
    dh, do = w1.shape[1], w2.shape[1]

    def body(x_r, w1_r, w2_r, v_r, o_r, *po):
        bf = jnp.bfloat16
        h = jnp.maximum(
            _dot(x_r[...].astype(bf), w1_r[...]) + v_r[0:1, :dh], 0.0)
        h = _dot(h.astype(bf), w2_r[...]) + v_r[1:2, :do]
        y = _ln(h, v_r[2:3, :do], v_r[3:4, :do])
        o_r[...] = y
        if want_packed:
            po[0][...] = _pack_bf16(y)

    out_specs = [pl.BlockSpec((T, do), lambda i: (i, 0))]
    out_shape = [jax.ShapeDtypeStruct((n, do), F32)]
    if want_packed:
        out_specs.append(pl.BlockSpec((T, do // 2), lambda i: (i, 0)))
        out_shape.append(jax.ShapeDtypeStruct((n, do // 2), jnp.int32))
    r = pl.pallas_call(
        body,
        grid=(n // T,),
        in_specs=[
            pl.BlockSpec((T, din), lambda i: (i, 0)),
            pl.BlockSpec((din, dh), lambda i: (0, 0)),
            pl.BlockSpec((dh, do), lambda i: (0, 0)),
            pl.BlockSpec((8, vecs.shape[1]), lambda i: (0, 0)),
        ],
        out_specs=out_specs if want_packed else out_specs[0],
        out_shape=out_shape if want_packed else out_shape[0],
        compiler_params=_CP,
    )(x, w1, w2, vecs)
    return r


def _edge_mlp(xs_arr, xs_off, xd_arr, xd_off, e, w1s, w1d, w1e, vecs, w2,
              n_edges, want_enew):
    nb = n_edges // T_EDGE
    has_xd = xd_arr is not None

    def body(*refs):
        if has_xd:
            xs_r, xd_r, e_r, w1s_r, w1d_r, w1e_r, w2_r, v_r, *outs = refs
        else:
            xs_r, e_r, w1s_r, w1e_r, w2_r, v_r, *outs = refs
        bf = jnp.bfloat16
        hd = D // 2
        lo, hi = _unpack_bf16(xs_r[...])
        acc = _dot(lo, w1s_r[:hd]) + _dot(hi, w1s_r[hd:])
        if has_xd:
            lo, hi = _unpack_bf16(xd_r[...])
            acc = acc + _dot(lo, w1d_r[:hd]) + _dot(hi, w1d_r[hd:])
        acc = acc + _dot(e_r[...].astype(bf), w1e_r[...]) + v_r[0:1, :]
        h = jnp.maximum(acc, 0.0)
        h = _dot(h.astype(bf), w2_r[...]) + v_r[1:2, :]
        m = _ln(h, v_r[2:3, :], v_r[3:4, :])
        outs[0][...] = m
        if want_enew:
            outs[1][...] = e_r[...] + m

    in_specs = [pl.BlockSpec((T_EDGE, D // 2), lambda i, o=xs_off: (i + o, 0))]
    inputs = [xs_arr]
    if has_xd:
        in_specs.append(
            pl.BlockSpec((T_EDGE, D // 2), lambda i, o=xd_off: (i + o, 0)))
        inputs.append(xd_arr)
    in_specs.append(pl.BlockSpec((T_EDGE, D), lambda i: (i, 0)))
    inputs.append(e)
    wspec = pl.BlockSpec((D, D), lambda i: (0, 0))
    in_specs += [wspec, wspec] if not has_xd else [wspec, wspec, wspec]
    inputs += [w1s, w1e] if not has_xd else [w1s, w1d, w1e]
    in_specs += [wspec, pl.BlockSpec((8, D), lambda i: (0, 0))]
    inputs += [w2, vecs]

    out_spec = pl.BlockSpec((T_EDGE, D), lambda i: (i, 0))
    out_shape = jax.ShapeDtypeStruct((n_edges, D), F32)
    if want_enew:
        out_specs, out_shapes = [out_spec, out_spec], [out_shape, out_shape]
    else:
        out_specs, out_shapes = out_spec, out_shape

    return pl.pallas_call(
        body,
        grid=(nb,),
        in_specs=in_specs,
        out_specs=out_specs,
        out_shape=out_shapes,
        compiler_params=_CP,
    )(*inputs)


def _node_mlp(x, aggs, w1x, w1a, vecs, w2, n_rows, T, want_packed=False):
    use_x = x is not None

    def body(*refs):
        refs = list(refs)
        p_r = refs.pop() if want_packed else None
        o_r = refs.pop()
        v_r = refs.pop()
        w2_r = refs.pop()
        x_r = refs.pop(0) if use_x else None
        agg_rs = [refs.pop(0) for _ in aggs]
        w1x_r = refs.pop(0) if use_x else None
        w1a_r = refs.pop(0) if aggs else None
        acc = v_r[0:1, :] * jnp.ones((T, 1), F32)
        if use_x:
            acc = acc + _dot(x_r[...], w1x_r[...])
        if agg_rs:
            a = agg_rs[0][...]
            for r in agg_rs[1:]:
                a = a + r[...]
            acc = acc + _dot(a, w1a_r[...])
        h = jnp.maximum(acc, 0.0)
        h = _dot(h, w2_r[...]) + v_r[1:2, :]
        m = _ln(h, v_r[2:3, :], v_r[3:4, :])
        y = (x_r[...] + m) if use_x else m
        o_r[...] = y
        if want_packed:
            p_r[...] = _pack_bf16(y)

    rspec = pl.BlockSpec((T, D), lambda i: (i, 0))
    wspec = pl.BlockSpec((D, D), lambda i: (0, 0))
    inputs, in_specs = [], []
    if use_x:
        inputs.append(x); in_specs.append(rspec)
    for a in aggs:
        inputs.append(a); in_specs.append(rspec)
    if use_x:
        inputs.append(w1x); in_specs.append(wspec)
    if aggs:
        inputs.append(w1a); in_specs.append(wspec)
    inputs += [w2, vecs]
    in_specs += [wspec, pl.BlockSpec((8, D), lambda i: (0, 0))]

    out_specs = [rspec]
    out_shape = [jax.ShapeDtypeStruct((n_rows, D), F32)]
    if want_packed:
        out_specs.append(pl.BlockSpec((T, D // 2), lambda i: (i, 0)))
        out_shape.append(jax.ShapeDtypeStruct((n_rows, D // 2), jnp.int32))
    return pl.pallas_call(
        body,
        grid=(n_rows // T,),
        in_specs=in_specs,
        out_specs=out_specs if want_packed else out_specs[0],
        out_shape=out_shape if want_packed else out_shape[0],
        compiler_params=_CP,
    )(*inputs)


def _final_mlp(x, featp, w1, vecs, w2):
    n = x.shape[0]
    dh = w1.shape[1]

    def body(x_r, f_r, w1_r, w2_r, v_r, o_r):
        h = jnp.maximum(_dot(x_r[...], w1_r[...]) + v_r[0:1, :], 0.0)
        o_r[...] = _dot(h, w2_r[...]) + v_r[1:2, :] + f_r[...]

    return pl.pallas_call(
        body,
        grid=(n // T_GRID,),
        in_specs=[
            pl.BlockSpec((T_GRID, D), lambda i: (i, 0)),
            pl.BlockSpec((T_GRID, dh), lambda i: (i, 0)),
            pl.BlockSpec((D, dh), lambda i: (0, 0)),
            pl.BlockSpec((dh, dh), lambda i: (0, 0)),
            pl.BlockSpec((8, dh), lambda i: (0, 0)),
        ],
        out_specs=pl.BlockSpec((T_GRID, dh), lambda i: (i, 0)),
        out_shape=jax.ShapeDtypeStruct((n, dh), F32),
        compiler_params=_CP,
    )(x, featp, w1, w2, vecs)


@functools.cache
def _mesh():
    return plsc.VectorSubcoreMesh(core_axis_name="c", subcore_axis_name="s",
                                  num_cores=NC, num_subcores=NS)


_CHUNK = 128


def _sc_gather(table, idx, n_out):
    return _sc_gather_raw(table, idx, n_out, table.shape[1])


def _sc_gather_raw(table, idx, n_out, w):
    bpw = n_out // NW
    nfull, rem = divmod(bpw, _CHUNK)
    cps = [_CHUNK] * nfull + ([rem] if rem else [])
    nch = len(cps)
    dt = table.dtype
    scratch = [pltpu.VMEM((bpw,), jnp.int32),
               pltpu.VMEM((_CHUNK, w), dt), pltpu.VMEM((_CHUNK, w), dt),
               pltpu.SemaphoreType.DMA, pltpu.SemaphoreType.DMA,
               pltpu.SemaphoreType.DMA, pltpu.SemaphoreType.DMA]

    @functools.partial(
        pl.kernel,
        out_type=jax.ShapeDtypeStruct((n_out, w), dt),
        mesh=_mesh(),
        scratch_types=scratch,
    )
    def k(table_hbm, idx_hbm, out_hbm, idx_v, r0, r1, sg0, sg1, sw0, sw1):
        wid = lax.axis_index("s") * NC + lax.axis_index("c")
        base = wid * bpw
        pltpu.sync_copy(idx_hbm.at[pl.ds(base, bpw)], idx_v)
        rows, sgs, sws = (r0, r1), (sg0, sg1), (sw0, sw1)

        def buf(i):
            b = i & 1
            sz = cps[i]
            return rows[b] if sz == _CHUNK else rows[b].at[pl.ds(0, sz)]

        def issue_g(i):
            sz = cps[i]
            return pltpu.async_copy(
                table_hbm.at[idx_v.at[pl.ds(i * _CHUNK, sz)]], buf(i),
                sgs[i & 1])

        def issue_w(i):
            sz = cps[i]
            return pltpu.async_copy(
                buf(i), out_hbm.at[pl.ds(base + i * _CHUNK, sz)], sws[i & 1])

        gd, wd = [None] * nch, [None] * nch
        gd[0] = issue_g(0)
        for i in range(nch):
            if i + 1 < nch:
                if i >= 1:
                    wd[i - 1].wait()
                gd[i + 1] = issue_g(i + 1)
            gd[i].wait()
            wd[i] = issue_w(i)
        wd[nch - 1].wait()
        if nch >= 2:
            wd[nch - 2].wait()

    return k(table, idx)


def _sc_scatter(m, dst, zeros, n_edges, n_nodes):
    dh = D // NC
    epw = n_edges // NS
    nch = epw // _CHUNK
    assert nch * _CHUNK == epw
    rps = n_nodes // NS
    scratch = [pltpu.VMEM((1, _CHUNK), jnp.int32), pltpu.VMEM((1, _CHUNK), jnp.int32),
               pltpu.VMEM((_CHUNK, dh), F32), pltpu.VMEM((_CHUNK, dh), F32),
               pltpu.SemaphoreType.DMA, pltpu.SemaphoreType.DMA,
               pltpu.SemaphoreType.DMA, pltpu.SemaphoreType.DMA,
               pltpu.SemaphoreType.DMA, pltpu.SemaphoreType.DMA,
               pltpu.VMEM_SHARED((n_nodes, dh), F32)]

    @functools.partial(
        pl.kernel,
        out_type=jax.ShapeDtypeStruct((n_nodes, D), F32),
        mesh=_mesh(),
        scratch_types=scratch,
    )
    def k(m_hbm, dst_hbm, z_hbm, out_hbm, i0, i1, m0, m1,
          si0, si1, sm0, sm1, ss0, ss1, acc):
        cid = lax.axis_index("c")
        sid = lax.axis_index("s")
        col = cid * dh
        pltpu.sync_copy(z_hbm, acc.at[pl.ds(sid * rps, rps)])
        plsc.subcore_barrier()
        base = sid * epw
        idxs, mbufs = (i0, i1), (m0, m1)
        sis, sms, sss = (si0, si1), (sm0, sm1), (ss0, ss1)

        def issue_l(i):
            b = i & 1
            off = base + i * _CHUNK
            di = pltpu.async_copy(dst_hbm.at[pl.ds(off, _CHUNK)],
                                  idxs[b].at[0], sis[b])
            dm = pltpu.async_copy(m_hbm.at[pl.ds(off, _CHUNK), pl.ds(col, dh)],
                                  mbufs[b], sms[b])
            return di, dm

        def issue_s(i):
            b = i & 1
            return pltpu.async_copy(mbufs[b], acc.at[idxs[b].at[0]], sss[b],
                                    add=True)

        ld, sd = [None] * nch, [None] * nch
        ld[0] = issue_l(0)
        for i in range(nch):
            if i + 1 < nch:
                if i >= 1:
                    sd[i - 1].wait()
                ld[i + 1] = issue_l(i + 1)
            ld[i][0].wait()
            ld[i][1].wait()
            sd[i] = issue_s(i)
        sd[nch - 1].wait()
        if nch >= 2:
            sd[nch - 2].wait()
        plsc.subcore_barrier()
        pltpu.sync_copy(acc.at[pl.ds(sid * rps, rps)],
                        out_hbm.at[pl.ds(sid * rps, rps), pl.ds(col, dh)])

    return k(m, dst, zeros)


def _pad_idx(a, n, fill_start, fill_mod):
    npad = n - a.shape[0]
    pad = fill_start + (jnp.arange(npad, dtype=jnp.int32) % fill_mod)
    return jnp.concatenate([a.astype(jnp.int32), pad])


def _pad2(a, rows, cols):
    return jnp.pad(a, ((0, rows - a.shape[0]), (0, cols - a.shape[1])))


def kernel(features, params, graph):
    f = features[0]
    P, G = params, graph

    fpad = _pad2(f, N_GRID, 128)
    zeros_lat = jnp.zeros((LAT_P // NS, D // NC), F32)
    zeros_grid = jnp.zeros((GRID_P // NS, D // NC), F32)

    enc_src = _pad_idx(G["enc_src"], EP_E, 0, N_GRID)
    enc_dst = _pad_idx(G["enc_dst"], EP_E, N_LAT, LAT_P - N_LAT)
    proc_srcdst = jnp.concatenate([
        _pad_idx(G["proc_src"], EP_P, 0, N_LAT),
        _pad_idx(G["proc_dst"], EP_P, 0, N_LAT)])
    proc_dst = _pad_idx(G["proc_dst"], EP_P, N_LAT, LAT_P - N_LAT)
    dec_srcdst = jnp.concatenate([
        _pad_idx(G["dec_src"], EP_E, 0, N_LAT),
        LAT_P + _pad_idx(G["dec_dst"], EP_E, 0, N_GRID)])
    dec_dst_s = _pad_idx(G["dec_dst"], EP_E, N_GRID, GRID_P - N_GRID)

    bfc = lambda a: a.astype(jnp.bfloat16)
    pne = P["node_encoder"]
    x_grid, x_grid_pk = _mlp_ln(fpad, bfc(_pad2(pne["w1"], 128, D)),
                                _vecs(pne["b1"], pne["b2"], pne["ln_g"],
                                      pne["ln_b"]),
                                bfc(pne["w2"]), T_GRID, want_packed=True)

    def enc_attr(p, attr, n_pad):
        return _mlp_ln(_pad2(attr, n_pad, 8), bfc(_pad2(p["w1"], 8, D)),
                       _vecs(p["b1"], p["b2"], p["ln_g"], p["ln_b"]),
                       bfc(p["w2"]), T_EDGE)

    e_enc = enc_attr(P["enc_edge_encoder"], G["enc_attr"], EP_E)
    ep = enc_attr(P["proc_edge_encoder"], G["proc_attr"], EP_P)
    ed = enc_attr(P["dec_edge_encoder"], G["dec_attr"], EP_E)

    def edge_w(p):
        w1 = p["w1"].astype(jnp.bfloat16)
        return (w1[:D], w1[D:2 * D], w1[2 * D:],
                _vecs(p["b1"], p["b2"], p["ln_g"], p["ln_b"]),
                p["w2"].astype(jnp.bfloat16))

    def node_w(p):
        w1 = p["w1"]
        return (w1[:D], w1[D:],
                _vecs(p["b1"], p["b2"], p["ln_g"], p["ln_b"]), p["w2"])

    w1s, _, w1e, vecs, w2 = edge_w(P["enc_block"]["edge"])
    xs = _sc_gather(x_grid_pk, enc_src, EP_E)
    m = _edge_mlp(xs, 0, None, 0, e_enc, w1s, None, w1e, vecs, w2, EP_E, False)
    agg = _sc_scatter(m, enc_dst, zeros_lat, EP_E, LAT_P)
    w1x, w1a, nvecs, nw2 = node_w(P["enc_block"]["node"])
    x_lat, x_lat_pk = _node_mlp(None, [agg], None, w1a, nvecs, nw2, LAT_P,
                                T_LAT, want_packed=True)
    x_grid, x_grid_pk = _node_mlp(x_grid, [], w1x, None, nvecs, nw2, N_GRID,
                                  T_GRID, want_packed=True)

    nb_p = EP_P // T_EDGE
    for bp in P["proc_blocks"]:
        w1s, w1d, w1e, vecs, w2 = edge_w(bp["edge"])
        rows = _sc_gather(x_lat_pk, proc_srcdst, 2 * EP_P)
        m, ep = _edge_mlp(rows, 0, rows, nb_p, ep, w1s, w1d, w1e, vecs, w2,
                          EP_P, True)
        agg = _sc_scatter(m, proc_dst, zeros_lat, EP_P, LAT_P)
        w1x, w1a, nvecs, nw2 = node_w(bp["node"])
        x_lat, x_lat_pk = _node_mlp(x_lat, [agg], w1x, w1a, nvecs, nw2, LAT_P,
                                    T_LAT, want_packed=True)

    w1s, w1d, w1e, vecs, w2 = edge_w(P["dec_block"]["edge"])
    table = jnp.concatenate([x_lat_pk, x_grid_pk])
    rows = _sc_gather(table, dec_srcdst, 2 * EP_E)
    nb_e = EP_E // T_EDGE
    m = _edge_mlp(rows, 0, rows, nb_e, ed, w1s, w1d, w1e, vecs, w2, EP_E, False)
    agg = _sc_scatter(m, dec_dst_s, zeros_grid, EP_E, GRID_P)
    w1x, w1a, nvecs, nw2 = node_w(P["dec_block"]["node"])
    x_grid = _node_mlp(x_grid, [agg], w1x, w1a, nvecs, nw2, N_GRID, T_GRID)

    pd = P["node_decoder"]
    out = _final_mlp(x_grid, fpad, pd["w1"],
                     _vecs(pd["b1"], jnp.pad(pd["b2"], (0, 128 - D_IN))),
                     _pad2(pd["w2"], 128, 128))
    return out[:, :D_IN][None]


# speedup vs baseline: 1.0479x; 1.0057x over previous
import functools

import jax
import jax.numpy as jnp
from jax import lax
from jax.experimental import pallas as pl
from jax.experimental.pallas import tpu as pltpu
from jax.experimental.pallas import tpu_sc as plsc

D = 256
NC, NS = 2, 16
NW = NC * NS
N_GRID, N_LAT, D_IN = 10000, 2562, 78
EP_P = 16384
EP_E = 30720
LAT_P = 2688
GRID_P = 10240
T_EDGE = 256
T_LAT = 128
T_GRID = 400
F32 = jnp.float32


def _vecs(*vs):
    w = max(v.shape[0] for v in vs)
    out = jnp.zeros((8, w), F32)
    for i, v in enumerate(vs):
        out = out.at[i, : v.shape[0]].set(v)
    return out


def _ln(h, g, b):
    mu = jnp.mean(h, axis=-1, keepdims=True)
    var = jnp.mean((h - mu) ** 2, axis=-1, keepdims=True)
    return (h - mu) * lax.rsqrt(var + 1e-5) * g + b


def _dot(a, b):
    return jnp.dot(a, b, preferred_element_type=F32)


def _pack_bf16(x):
    lo = lax.bitcast_convert_type(x[:, :D // 2].astype(jnp.bfloat16),
                                  jnp.uint16).astype(jnp.uint32)
    hi = lax.bitcast_convert_type(x[:, D // 2:].astype(jnp.bfloat16),
                                  jnp.uint16).astype(jnp.uint32)
    return lax.bitcast_convert_type(lo | (hi << 16), jnp.int32)


def _unpack_bf16(xi):
    xu = lax.bitcast_convert_type(xi, jnp.uint32)
    lo = lax.bitcast_convert_type((xu & 0xFFFF).astype(jnp.uint16), jnp.bfloat16)
    hi = lax.bitcast_convert_type((xu >> 16).astype(jnp.uint16), jnp.bfloat16)
    return lo, hi


_CP = pltpu.CompilerParams(dimension_semantics=("parallel",))


def _mlp_ln(x, w1, vecs, w2, T, want_packed=False):
    n, din = x.shape
    dh, do = w1.shape[1], w2.shape[1]

    def body(x_r, w1_r, w2_r, v_r, o_r, *po):
        h = jnp.maximum(_dot(x_r[...], w1_r[...]) + v_r[0:1, :dh], 0.0)
        h = _dot(h, w2_r[...]) + v_r[1:2, :do]
        y = _ln(h, v_r[2:3, :do], v_r[3:4, :do])
        o_r[...] = y
        if want_packed:
            po[0][...] = _pack_bf16(y)

    out_specs = [pl.BlockSpec((T, do), lambda i: (i, 0))]
    out_shape = [jax.ShapeDtypeStruct((n, do), F32)]
    if want_packed:
        out_specs.append(pl.BlockSpec((T, do // 2), lambda i: (i, 0)))
        out_shape.append(jax.ShapeDtypeStruct((n, do // 2), jnp.int32))
    r = pl.pallas_call(
        body,
        grid=(n // T,),
        in_specs=[
            pl.BlockSpec((T, din), lambda i: (i, 0)),
            pl.BlockSpec((din, dh), lambda i: (0, 0)),
            pl.BlockSpec((dh, do), lambda i: (0, 0)),
            pl.BlockSpec((8, vecs.shape[1]), lambda i: (0, 0)),
        ],
        out_specs=out_specs if want_packed else out_specs[0],
        out_shape=out_shape if want_packed else out_shape[0],
        compiler_params=_CP,
    )(x, w1, w2, vecs)
    return r


def _edge_mlp(xs_arr, xs_off, xd_arr, xd_off, e, w1s, w1d, w1e, vecs, w2,
              n_edges, want_enew):
    nb = n_edges // T_EDGE
    has_xd = xd_arr is not None

    def body(*refs):
        if has_xd:
            xs_r, xd_r, e_r, w1s_r, w1d_r, w1e_r, w2_r, v_r, *outs = refs
        else:
            xs_r, e_r, w1s_r, w1e_r, w2_r, v_r, *outs = refs
        bf = jnp.bfloat16
        hd = D // 2
        lo, hi = _unpack_bf16(xs_r[...])
        acc = _dot(lo, w1s_r[:hd]) + _dot(hi, w1s_r[hd:])
        if has_xd:
            lo, hi = _unpack_bf16(xd_r[...])
            acc = acc + _dot(lo, w1d_r[:hd]) + _dot(hi, w1d_r[hd:])
        acc = acc + _dot(e_r[...].astype(bf), w1e_r[...]) + v_r[0:1, :]
        h = jnp.maximum(acc, 0.0)
        h = _dot(h.astype(bf), w2_r[...]) + v_r[1:2, :]
        m = _ln(h, v_r[2:3, :], v_r[3:4, :])
        outs[0][...] = m
        if want_enew:
            outs[1][...] = e_r[...] + m

    in_specs = [pl.BlockSpec((T_EDGE, D // 2), lambda i, o=xs_off: (i + o, 0))]
    inputs = [xs_arr]
    if has_xd:
        in_specs.append(
            pl.BlockSpec((T_EDGE, D // 2), lambda i, o=xd_off: (i + o, 0)))
        inputs.append(xd_arr)
    in_specs.append(pl.BlockSpec((T_EDGE, D), lambda i: (i, 0)))
    inputs.append(e)
    wspec = pl.BlockSpec((D, D), lambda i: (0, 0))
    in_specs += [wspec, wspec] if not has_xd else [wspec, wspec, wspec]
    inputs += [w1s, w1e] if not has_xd else [w1s, w1d, w1e]
    in_specs += [wspec, pl.BlockSpec((8, D), lambda i: (0, 0))]
    inputs += [w2, vecs]

    out_spec = pl.BlockSpec((T_EDGE, D), lambda i: (i, 0))
    out_shape = jax.ShapeDtypeStruct((n_edges, D), F32)
    if want_enew:
        out_specs, out_shapes = [out_spec, out_spec], [out_shape, out_shape]
    else:
        out_specs, out_shapes = out_spec, out_shape

    return pl.pallas_call(
        body,
        grid=(nb,),
        in_specs=in_specs,
        out_specs=out_specs,
        out_shape=out_shapes,
        compiler_params=_CP,
    )(*inputs)


def _node_mlp(x, aggs, w1x, w1a, vecs, w2, n_rows, T, want_packed=False):
    use_x = x is not None

    def body(*refs):
        refs = list(refs)
        p_r = refs.pop() if want_packed else None
        o_r = refs.pop()
        v_r = refs.pop()
        w2_r = refs.pop()
        x_r = refs.pop(0) if use_x else None
        agg_rs = [refs.pop(0) for _ in aggs]
        w1x_r = refs.pop(0) if use_x else None
        w1a_r = refs.pop(0) if aggs else None
        acc = v_r[0:1, :] * jnp.ones((T, 1), F32)
        if use_x:
            acc = acc + _dot(x_r[...], w1x_r[...])
        if agg_rs:
            a = agg_rs[0][...]
            for r in agg_rs[1:]:
                a = a + r[...]
            acc = acc + _dot(a, w1a_r[...])
        h = jnp.maximum(acc, 0.0)
        h = _dot(h, w2_r[...]) + v_r[1:2, :]
        m = _ln(h, v_r[2:3, :], v_r[3:4, :])
        y = (x_r[...] + m) if use_x else m
        o_r[...] = y
        if want_packed:
            p_r[...] = _pack_bf16(y)

    rspec = pl.BlockSpec((T, D), lambda i: (i, 0))
    wspec = pl.BlockSpec((D, D), lambda i: (0, 0))
    inputs, in_specs = [], []
    if use_x:
        inputs.append(x); in_specs.append(rspec)
    for a in aggs:
        inputs.append(a); in_specs.append(rspec)
    if use_x:
        inputs.append(w1x); in_specs.append(wspec)
    if aggs:
        inputs.append(w1a); in_specs.append(wspec)
    inputs += [w2, vecs]
    in_specs += [wspec, pl.BlockSpec((8, D), lambda i: (0, 0))]

    out_specs = [rspec]
    out_shape = [jax.ShapeDtypeStruct((n_rows, D), F32)]
    if want_packed:
        out_specs.append(pl.BlockSpec((T, D // 2), lambda i: (i, 0)))
        out_shape.append(jax.ShapeDtypeStruct((n_rows, D // 2), jnp.int32))
    return pl.pallas_call(
        body,
        grid=(n_rows // T,),
        in_specs=in_specs,
        out_specs=out_specs if want_packed else out_specs[0],
        out_shape=out_shape if want_packed else out_shape[0],
        compiler_params=_CP,
    )(*inputs)


def _final_mlp(x, featp, w1, vecs, w2):
    n = x.shape[0]
    dh = w1.shape[1]

    def body(x_r, f_r, w1_r, w2_r, v_r, o_r):
        h = jnp.maximum(_dot(x_r[...], w1_r[...]) + v_r[0:1, :], 0.0)
        o_r[...] = _dot(h, w2_r[...]) + v_r[1:2, :] + f_r[...]

    return pl.pallas_call(
        body,
        grid=(n // T_GRID,),
        in_specs=[
            pl.BlockSpec((T_GRID, D), lambda i: (i, 0)),
            pl.BlockSpec((T_GRID, dh), lambda i: (i, 0)),
            pl.BlockSpec((D, dh), lambda i: (0, 0)),
            pl.BlockSpec((dh, dh), lambda i: (0, 0)),
            pl.BlockSpec((8, dh), lambda i: (0, 0)),
        ],
        out_specs=pl.BlockSpec((T_GRID, dh), lambda i: (i, 0)),
        out_shape=jax.ShapeDtypeStruct((n, dh), F32),
        compiler_params=_CP,
    )(x, featp, w1, w2, vecs)


@functools.cache
def _mesh():
    return plsc.VectorSubcoreMesh(core_axis_name="c", subcore_axis_name="s",
                                  num_cores=NC, num_subcores=NS)


_CHUNK = 128


def _sc_gather(table, idx, n_out):
    return _sc_gather_raw(table, idx, n_out, table.shape[1])


def _sc_gather_raw(table, idx, n_out, w):
    bpw = n_out // NW
    nfull, rem = divmod(bpw, _CHUNK)
    cps = [_CHUNK] * nfull + ([rem] if rem else [])
    nch = len(cps)
    dt = table.dtype
    scratch = [pltpu.VMEM((bpw,), jnp.int32),
               pltpu.VMEM((_CHUNK, w), dt), pltpu.VMEM((_CHUNK, w), dt),
               pltpu.SemaphoreType.DMA, pltpu.SemaphoreType.DMA,
               pltpu.SemaphoreType.DMA, pltpu.SemaphoreType.DMA]

    @functools.partial(
        pl.kernel,
        out_type=jax.ShapeDtypeStruct((n_out, w), dt),
        mesh=_mesh(),
        scratch_types=scratch,
    )
    def k(table_hbm, idx_hbm, out_hbm, idx_v, r0, r1, sg0, sg1, sw0, sw1):
        wid = lax.axis_index("s") * NC + lax.axis_index("c")
        base = wid * bpw
        pltpu.sync_copy(idx_hbm.at[pl.ds(base, bpw)], idx_v)
        rows, sgs, sws = (r0, r1), (sg0, sg1), (sw0, sw1)

        def buf(i):
            b = i & 1
            sz = cps[i]
            return rows[b] if sz == _CHUNK else rows[b].at[pl.ds(0, sz)]

        def issue_g(i):
            sz = cps[i]
            return pltpu.async_copy(
                table_hbm.at[idx_v.at[pl.ds(i * _CHUNK, sz)]], buf(i),
                sgs[i & 1])

        def issue_w(i):
            sz = cps[i]
            return pltpu.async_copy(
                buf(i), out_hbm.at[pl.ds(base + i * _CHUNK, sz)], sws[i & 1])

        gd, wd = [None] * nch, [None] * nch
        gd[0] = issue_g(0)
        for i in range(nch):
            if i + 1 < nch:
                if i >= 1:
                    wd[i - 1].wait()
                gd[i + 1] = issue_g(i + 1)
            gd[i].wait()
            wd[i] = issue_w(i)
        wd[nch - 1].wait()
        if nch >= 2:
            wd[nch - 2].wait()

    return k(table, idx)


def _sc_scatter(m, dst, zeros, n_edges, n_nodes):
    dh = D // NC
    epw = n_edges // NS
    nch = epw // _CHUNK
    assert nch * _CHUNK == epw
    rps = n_nodes // NS
    scratch = [pltpu.VMEM((1, _CHUNK), jnp.int32), pltpu.VMEM((1, _CHUNK), jnp.int32),
               pltpu.VMEM((_CHUNK, dh), F32), pltpu.VMEM((_CHUNK, dh), F32),
               pltpu.SemaphoreType.DMA, pltpu.SemaphoreType.DMA,
               pltpu.SemaphoreType.DMA, pltpu.SemaphoreType.DMA,
               pltpu.SemaphoreType.DMA, pltpu.SemaphoreType.DMA,
               pltpu.VMEM_SHARED((n_nodes, dh), F32)]

    @functools.partial(
        pl.kernel,
        out_type=jax.ShapeDtypeStruct((n_nodes, D), F32),
        mesh=_mesh(),
        scratch_types=scratch,
    )
    def k(m_hbm, dst_hbm, z_hbm, out_hbm, i0, i1, m0, m1,
          si0, si1, sm0, sm1, ss0, ss1, acc):
        cid = lax.axis_index("c")
        sid = lax.axis_index("s")
        col = cid * dh
        pltpu.sync_copy(z_hbm, acc.at[pl.ds(sid * rps, rps)])
        plsc.subcore_barrier()
        base = sid * epw
        idxs, mbufs = (i0, i1), (m0, m1)
        sis, sms, sss = (si0, si1), (sm0, sm1), (ss0, ss1)

        def issue_l(i):
            b = i & 1
            off = base + i * _CHUNK
            di = pltpu.async_copy(dst_hbm.at[pl.ds(off, _CHUNK)],
                                  idxs[b].at[0], sis[b])
            dm = pltpu.async_copy(m_hbm.at[pl.ds(off, _CHUNK), pl.ds(col, dh)],
                                  mbufs[b], sms[b])
            return di, dm

        def issue_s(i):
            b = i & 1
            return pltpu.async_copy(mbufs[b], acc.at[idxs[b].at[0]], sss[b],
                                    add=True)

        ld, sd = [None] * nch, [None] * nch
        ld[0] = issue_l(0)
        for i in range(nch):
            if i + 1 < nch:
                if i >= 1:
                    sd[i - 1].wait()
                ld[i + 1] = issue_l(i + 1)
            ld[i][0].wait()
            ld[i][1].wait()
            sd[i] = issue_s(i)
        sd[nch - 1].wait()
        if nch >= 2:
            sd[nch - 2].wait()
        plsc.subcore_barrier()
        pltpu.sync_copy(acc.at[pl.ds(sid * rps, rps)],
                        out_hbm.at[pl.ds(sid * rps, rps), pl.ds(col, dh)])

    return k(m, dst, zeros)


def _pad_idx(a, n, fill_start, fill_mod):
    npad = n - a.shape[0]
    pad = fill_start + (jnp.arange(npad, dtype=jnp.int32) % fill_mod)
    return jnp.concatenate([a.astype(jnp.int32), pad])


def _pad2(a, rows, cols):
    return jnp.pad(a, ((0, rows - a.shape[0]), (0, cols - a.shape[1])))


def kernel(features, params, graph):
    f = features[0]
    P, G = params, graph

    fpad = _pad2(f, N_GRID, 128)
    zeros_lat = jnp.zeros((LAT_P // NS, D // NC), F32)
    zeros_grid = jnp.zeros((GRID_P // NS, D // NC), F32)

    enc_src = _pad_idx(G["enc_src"], EP_E, 0, N_GRID)
    enc_dst = _pad_idx(G["enc_dst"], EP_E, N_LAT, LAT_P - N_LAT)
    proc_srcdst = jnp.concatenate([
        _pad_idx(G["proc_src"], EP_P, 0, N_LAT),
        _pad_idx(G["proc_dst"], EP_P, 0, N_LAT)])
    proc_dst = _pad_idx(G["proc_dst"], EP_P, N_LAT, LAT_P - N_LAT)
    dec_srcdst = jnp.concatenate([
        _pad_idx(G["dec_src"], EP_E, 0, N_LAT),
        LAT_P + _pad_idx(G["dec_dst"], EP_E, 0, N_GRID)])
    dec_dst_s = _pad_idx(G["dec_dst"], EP_E, N_GRID, GRID_P - N_GRID)

    pne = P["node_encoder"]
    x_grid, x_grid_pk = _mlp_ln(fpad, _pad2(pne["w1"], 128, D),
                                _vecs(pne["b1"], pne["b2"], pne["ln_g"],
                                      pne["ln_b"]),
                                pne["w2"], T_GRID, want_packed=True)

    def enc_attr(p, attr, n_pad):
        return _mlp_ln(_pad2(attr, n_pad, 128), _pad2(p["w1"], 128, D),
                       _vecs(p["b1"], p["b2"], p["ln_g"], p["ln_b"]),
                       p["w2"], T_EDGE)

    e_enc = enc_attr(P["enc_edge_encoder"], G["enc_attr"], EP_E)
    ep = enc_attr(P["proc_edge_encoder"], G["proc_attr"], EP_P)
    ed = enc_attr(P["dec_edge_encoder"], G["dec_attr"], EP_E)

    def edge_w(p):
        w1 = p["w1"].astype(jnp.bfloat16)
        return (w1[:D], w1[D:2 * D], w1[2 * D:],
                _vecs(p["b1"], p["b2"], p["ln_g"], p["ln_b"]),
                p["w2"].astype(jnp.bfloat16))

    def node_w(p):
        w1 = p["w1"]
        return (w1[:D], w1[D:],
                _vecs(p["b1"], p["b2"], p["ln_g"], p["ln_b"]), p["w2"])

    w1s, _, w1e, vecs, w2 = edge_w(P["enc_block"]["edge"])
    xs = _sc_gather(x_grid_pk, enc_src, EP_E)
    m = _edge_mlp(xs, 0, None, 0, e_enc, w1s, None, w1e, vecs, w2, EP_E, False)
    agg = _sc_scatter(m, enc_dst, zeros_lat, EP_E, LAT_P)
    w1x, w1a, nvecs, nw2 = node_w(P["enc_block"]["node"])
    x_lat, x_lat_pk = _node_mlp(None, [agg], None, w1a, nvecs, nw2, LAT_P,
                                T_LAT, want_packed=True)
    x_grid, x_grid_pk = _node_mlp(x_grid, [], w1x, None, nvecs, nw2, N_GRID,
                                  T_GRID, want_packed=True)

    nb_p = EP_P // T_EDGE
    for bp in P["proc_blocks"]:
        w1s, w1d, w1e, vecs, w2 = edge_w(bp["edge"])
        rows = _sc_gather(x_lat_pk, proc_srcdst, 2 * EP_P)
        m, ep = _edge_mlp(rows, 0, rows, nb_p, ep, w1s, w1d, w1e, vecs, w2,
                          EP_P, True)
        agg = _sc_scatter(m, proc_dst, zeros_lat, EP_P, LAT_P)
        w1x, w1a, nvecs, nw2 = node_w(bp["node"])
        x_lat, x_lat_pk = _node_mlp(x_lat, [agg], w1x, w1a, nvecs, nw2, LAT_P,
                                    T_LAT, want_packed=True)

    w1s, w1d, w1e, vecs, w2 = edge_w(P["dec_block"]["edge"])
    table = jnp.concatenate([x_lat_pk, x_grid_pk])
    rows = _sc_gather(table, dec_srcdst, 2 * EP_E)
    nb_e = EP_E // T_EDGE
    m = _edge_mlp(rows, 0, rows, nb_e, ed, w1s, w1d, w1e, vecs, w2, EP_E, False)
    agg = _sc_scatter(m, dec_dst_s, zeros_grid, EP_E, GRID_P)
    w1x, w1a, nvecs, nw2 = node_w(P["dec_block"]["node"])
    x_grid = _node_mlp(x_grid, [agg], w1x, w1a, nvecs, nw2, N_GRID, T_GRID)

    pd = P["node_decoder"]
    out = _final_mlp(x_grid, fpad, pd["w1"],
                     _vecs(pd["b1"], jnp.pad(pd["b2"], (0, 128 - D_IN))),
                     _pad2(pd["w2"], 128, 128))
    return out[:, :D_IN][None]


# speedup vs baseline: 1.3099x; 1.2501x over previous
import functools

import jax
import jax.numpy as jnp
from jax import lax
from jax.experimental import pallas as pl
from jax.experimental.pallas import tpu as pltpu
from jax.experimental.pallas import tpu_sc as plsc

D = 256
NC, NS = 2, 16
NW = NC * NS
N_GRID, N_LAT, D_IN = 10000, 2562, 78
EP_P = 16384
EP_E = 30720
LAT_P = 2688
GRID_P = 10240
T_EDGE = 512
T_LAT = 128
T_GRID = 400
F32 = jnp.float32


def _vecs(*vs):
    w = max(v.shape[0] for v in vs)
    out = jnp.zeros((8, w), F32)
    for i, v in enumerate(vs):
        out = out.at[i, : v.shape[0]].set(v)
    return out


def _ln(h, g, b):
    mu = jnp.mean(h, axis=-1, keepdims=True)
    var = jnp.mean((h - mu) ** 2, axis=-1, keepdims=True)
    return (h - mu) * lax.rsqrt(var + 1e-5) * g + b


def _dot(a, b):
    return jnp.dot(a, b, preferred_element_type=F32)


def _pack_bf16(x):
    lo = lax.bitcast_convert_type(x[:, :D // 2].astype(jnp.bfloat16),
                                  jnp.uint16).astype(jnp.uint32)
    hi = lax.bitcast_convert_type(x[:, D // 2:].astype(jnp.bfloat16),
                                  jnp.uint16).astype(jnp.uint32)
    return lax.bitcast_convert_type(lo | (hi << 16), jnp.int32)


def _unpack_bf16(xi):
    xu = lax.bitcast_convert_type(xi, jnp.uint32)
    lo = lax.bitcast_convert_type((xu & 0xFFFF).astype(jnp.uint16), jnp.bfloat16)
    hi = lax.bitcast_convert_type((xu >> 16).astype(jnp.uint16), jnp.bfloat16)
    return lo, hi


_CP = pltpu.CompilerParams(dimension_semantics=("parallel",))


def _mlp_ln(x, w1, vecs, w2, T, want_packed=False):
    n, din = x.shape
    dh, do = w1.shape[1], w2.shape[1]

    def body(x_r, w1_r, w2_r, v_r, o_r, *po):
        h = jnp.maximum(_dot(x_r[...], w1_r[...]) + v_r[0:1, :dh], 0.0)
        h = _dot(h, w2_r[...]) + v_r[1:2, :do]
        y = _ln(h, v_r[2:3, :do], v_r[3:4, :do])
        o_r[...] = y
        if want_packed:
            po[0][...] = _pack_bf16(y)

    out_specs = [pl.BlockSpec((T, do), lambda i: (i, 0))]
    out_shape = [jax.ShapeDtypeStruct((n, do), F32)]
    if want_packed:
        out_specs.append(pl.BlockSpec((T, do // 2), lambda i: (i, 0)))
        out_shape.append(jax.ShapeDtypeStruct((n, do // 2), jnp.int32))
    r = pl.pallas_call(
        body,
        grid=(n // T,),
        in_specs=[
            pl.BlockSpec((T, din), lambda i: (i, 0)),
            pl.BlockSpec((din, dh), lambda i: (0, 0)),
            pl.BlockSpec((dh, do), lambda i: (0, 0)),
            pl.BlockSpec((8, vecs.shape[1]), lambda i: (0, 0)),
        ],
        out_specs=out_specs if want_packed else out_specs[0],
        out_shape=out_shape if want_packed else out_shape[0],
        compiler_params=_CP,
    )(x, w1, w2, vecs)
    return r


def _edge_mlp(xs_arr, xs_off, xd_arr, xd_off, e, w1s, w1d, w1e, vecs, w2,
              n_edges, want_enew):
    nb = n_edges // T_EDGE
    has_xd = xd_arr is not None

    def body(*refs):
        if has_xd:
            xs_r, xd_r, e_r, w1s_r, w1d_r, w1e_r, w2_r, v_r, *outs = refs
        else:
            xs_r, e_r, w1s_r, w1e_r, w2_r, v_r, *outs = refs
        bf = jnp.bfloat16
        hd = D // 2
        lo, hi = _unpack_bf16(xs_r[...])
        acc = _dot(lo, w1s_r[:hd]) + _dot(hi, w1s_r[hd:])
        if has_xd:
            lo, hi = _unpack_bf16(xd_r[...])
            acc = acc + _dot(lo, w1d_r[:hd]) + _dot(hi, w1d_r[hd:])
        acc = acc + _dot(e_r[...].astype(bf), w1e_r[...]) + v_r[0:1, :]
        h = jnp.maximum(acc, 0.0)
        h = _dot(h.astype(bf), w2_r[...]) + v_r[1:2, :]
        m = _ln(h, v_r[2:3, :], v_r[3:4, :])
        outs[0][...] = m
        if want_enew:
            outs[1][...] = e_r[...] + m

    in_specs = [pl.BlockSpec((T_EDGE, D // 2), lambda i, o=xs_off: (i + o, 0))]
    inputs = [xs_arr]
    if has_xd:
        in_specs.append(
            pl.BlockSpec((T_EDGE, D // 2), lambda i, o=xd_off: (i + o, 0)))
        inputs.append(xd_arr)
    in_specs.append(pl.BlockSpec((T_EDGE, D), lambda i: (i, 0)))
    inputs.append(e)
    wspec = pl.BlockSpec((D, D), lambda i: (0, 0))
    in_specs += [wspec, wspec] if not has_xd else [wspec, wspec, wspec]
    inputs += [w1s, w1e] if not has_xd else [w1s, w1d, w1e]
    in_specs += [wspec, pl.BlockSpec((8, D), lambda i: (0, 0))]
    inputs += [w2, vecs]

    out_spec = pl.BlockSpec((T_EDGE, D), lambda i: (i, 0))
    out_shape = jax.ShapeDtypeStruct((n_edges, D), F32)
    if want_enew:
        out_specs, out_shapes = [out_spec, out_spec], [out_shape, out_shape]
    else:
        out_specs, out_shapes = out_spec, out_shape

    return pl.pallas_call(
        body,
        grid=(nb,),
        in_specs=in_specs,
        out_specs=out_specs,
        out_shape=out_shapes,
        compiler_params=_CP,
    )(*inputs)


def _node_mlp(x, aggs, w1x, w1a, vecs, w2, n_rows, T, want_packed=False):
    use_x = x is not None

    def body(*refs):
        refs = list(refs)
        p_r = refs.pop() if want_packed else None
        o_r = refs.pop()
        v_r = refs.pop()
        w2_r = refs.pop()
        x_r = refs.pop(0) if use_x else None
        agg_rs = [refs.pop(0) for _ in aggs]
        w1x_r = refs.pop(0) if use_x else None
        w1a_r = refs.pop(0) if aggs else None
        acc = v_r[0:1, :] * jnp.ones((T, 1), F32)
        if use_x:
            acc = acc + _dot(x_r[...], w1x_r[...])
        if agg_rs:
            a = agg_rs[0][...]
            for r in agg_rs[1:]:
                a = a + r[...]
            acc = acc + _dot(a, w1a_r[...])
        h = jnp.maximum(acc, 0.0)
        h = _dot(h, w2_r[...]) + v_r[1:2, :]
        m = _ln(h, v_r[2:3, :], v_r[3:4, :])
        y = (x_r[...] + m) if use_x else m
        o_r[...] = y
        if want_packed:
            p_r[...] = _pack_bf16(y)

    rspec = pl.BlockSpec((T, D), lambda i: (i, 0))
    wspec = pl.BlockSpec((D, D), lambda i: (0, 0))
    inputs, in_specs = [], []
    if use_x:
        inputs.append(x); in_specs.append(rspec)
    for a in aggs:
        inputs.append(a); in_specs.append(rspec)
    if use_x:
        inputs.append(w1x); in_specs.append(wspec)
    if aggs:
        inputs.append(w1a); in_specs.append(wspec)
    inputs += [w2, vecs]
    in_specs += [wspec, pl.BlockSpec((8, D), lambda i: (0, 0))]

    out_specs = [rspec]
    out_shape = [jax.ShapeDtypeStruct((n_rows, D), F32)]
    if want_packed:
        out_specs.append(pl.BlockSpec((T, D // 2), lambda i: (i, 0)))
        out_shape.append(jax.ShapeDtypeStruct((n_rows, D // 2), jnp.int32))
    return pl.pallas_call(
        body,
        grid=(n_rows // T,),
        in_specs=in_specs,
        out_specs=out_specs if want_packed else out_specs[0],
        out_shape=out_shape if want_packed else out_shape[0],
        compiler_params=_CP,
    )(*inputs)


def _final_mlp(x, featp, w1, vecs, w2):
    n = x.shape[0]
    dh = w1.shape[1]

    def body(x_r, f_r, w1_r, w2_r, v_r, o_r):
        h = jnp.maximum(_dot(x_r[...], w1_r[...]) + v_r[0:1, :], 0.0)
        o_r[...] = _dot(h, w2_r[...]) + v_r[1:2, :] + f_r[...]

    return pl.pallas_call(
        body,
        grid=(n // T_GRID,),
        in_specs=[
            pl.BlockSpec((T_GRID, D), lambda i: (i, 0)),
            pl.BlockSpec((T_GRID, dh), lambda i: (i, 0)),
            pl.BlockSpec((D, dh), lambda i: (0, 0)),
            pl.BlockSpec((dh, dh), lambda i: (0, 0)),
            pl.BlockSpec((8, dh), lambda i: (0, 0)),
        ],
        out_specs=pl.BlockSpec((T_GRID, dh), lambda i: (i, 0)),
        out_shape=jax.ShapeDtypeStruct((n, dh), F32),
        compiler_params=_CP,
    )(x, featp, w1, w2, vecs)


@functools.cache
def _mesh():
    return plsc.VectorSubcoreMesh(core_axis_name="c", subcore_axis_name="s",
                                  num_cores=NC, num_subcores=NS)


_CHUNK = 128


def _sc_gather(table, idx, n_out):
    return _sc_gather_raw(table, idx, n_out, table.shape[1])


def _sc_gather_raw(table, idx, n_out, w):
    bpw = n_out // NW
    nfull, rem = divmod(bpw, _CHUNK)
    cps = [_CHUNK] * nfull + ([rem] if rem else [])
    nch = len(cps)
    dt = table.dtype
    scratch = [pltpu.VMEM((bpw,), jnp.int32),
               pltpu.VMEM((_CHUNK, w), dt), pltpu.VMEM((_CHUNK, w), dt),
               pltpu.SemaphoreType.DMA, pltpu.SemaphoreType.DMA,
               pltpu.SemaphoreType.DMA, pltpu.SemaphoreType.DMA]

    @functools.partial(
        pl.kernel,
        out_type=jax.ShapeDtypeStruct((n_out, w), dt),
        mesh=_mesh(),
        scratch_types=scratch,
    )
    def k(table_hbm, idx_hbm, out_hbm, idx_v, r0, r1, sg0, sg1, sw0, sw1):
        wid = lax.axis_index("s") * NC + lax.axis_index("c")
        base = wid * bpw
        pltpu.sync_copy(idx_hbm.at[pl.ds(base, bpw)], idx_v)
        rows, sgs, sws = (r0, r1), (sg0, sg1), (sw0, sw1)

        def buf(i):
            b = i & 1
            sz = cps[i]
            return rows[b] if sz == _CHUNK else rows[b].at[pl.ds(0, sz)]

        def issue_g(i):
            sz = cps[i]
            return pltpu.async_copy(
                table_hbm.at[idx_v.at[pl.ds(i * _CHUNK, sz)]], buf(i),
                sgs[i & 1])

        def issue_w(i):
            sz = cps[i]
            return pltpu.async_copy(
                buf(i), out_hbm.at[pl.ds(base + i * _CHUNK, sz)], sws[i & 1])

        gd, wd = [None] * nch, [None] * nch
        gd[0] = issue_g(0)
        for i in range(nch):
            if i + 1 < nch:
                if i >= 1:
                    wd[i - 1].wait()
                gd[i + 1] = issue_g(i + 1)
            gd[i].wait()
            wd[i] = issue_w(i)
        wd[nch - 1].wait()
        if nch >= 2:
            wd[nch - 2].wait()

    return k(table, idx)


def _sc_scatter(m, dst, zeros, n_edges, n_nodes):
    dh = D // NC
    epw = n_edges // NS
    nch = epw // _CHUNK
    assert nch * _CHUNK == epw
    rps = n_nodes // NS
    scratch = [pltpu.VMEM((1, _CHUNK), jnp.int32), pltpu.VMEM((1, _CHUNK), jnp.int32),
               pltpu.VMEM((_CHUNK, dh), F32), pltpu.VMEM((_CHUNK, dh), F32),
               pltpu.SemaphoreType.DMA, pltpu.SemaphoreType.DMA,
               pltpu.SemaphoreType.DMA, pltpu.SemaphoreType.DMA,
               pltpu.SemaphoreType.DMA, pltpu.SemaphoreType.DMA,
               pltpu.VMEM_SHARED((n_nodes, dh), F32)]

    @functools.partial(
        pl.kernel,
        out_type=jax.ShapeDtypeStruct((n_nodes, D), F32),
        mesh=_mesh(),
        scratch_types=scratch,
    )
    def k(m_hbm, dst_hbm, z_hbm, out_hbm, i0, i1, m0, m1,
          si0, si1, sm0, sm1, ss0, ss1, acc):
        cid = lax.axis_index("c")
        sid = lax.axis_index("s")
        col = cid * dh
        pltpu.sync_copy(z_hbm, acc.at[pl.ds(sid * rps, rps)])
        plsc.subcore_barrier()
        base = sid * epw
        idxs, mbufs = (i0, i1), (m0, m1)
        sis, sms, sss = (si0, si1), (sm0, sm1), (ss0, ss1)

        def issue_l(i):
            b = i & 1
            off = base + i * _CHUNK
            di = pltpu.async_copy(dst_hbm.at[pl.ds(off, _CHUNK)],
                                  idxs[b].at[0], sis[b])
            dm = pltpu.async_copy(m_hbm.at[pl.ds(off, _CHUNK), pl.ds(col, dh)],
                                  mbufs[b], sms[b])
            return di, dm

        def issue_s(i):
            b = i & 1
            return pltpu.async_copy(mbufs[b], acc.at[idxs[b].at[0]], sss[b],
                                    add=True)

        ld, sd = [None] * nch, [None] * nch
        ld[0] = issue_l(0)
        for i in range(nch):
            if i + 1 < nch:
                if i >= 1:
                    sd[i - 1].wait()
                ld[i + 1] = issue_l(i + 1)
            ld[i][0].wait()
            ld[i][1].wait()
            sd[i] = issue_s(i)
        sd[nch - 1].wait()
        if nch >= 2:
            sd[nch - 2].wait()
        plsc.subcore_barrier()
        pltpu.sync_copy(acc.at[pl.ds(sid * rps, rps)],
                        out_hbm.at[pl.ds(sid * rps, rps), pl.ds(col, dh)])

    return k(m, dst, zeros)


def _pad_idx(a, n, fill_start, fill_mod):
    npad = n - a.shape[0]
    pad = fill_start + (jnp.arange(npad, dtype=jnp.int32) % fill_mod)
    return jnp.concatenate([a.astype(jnp.int32), pad])


def _pad2(a, rows, cols):
    return jnp.pad(a, ((0, rows - a.shape[0]), (0, cols - a.shape[1])))


def kernel(features, params, graph):
    f = features[0]
    P, G = params, graph

    fpad = _pad2(f, N_GRID, 128)
    zeros_lat = jnp.zeros((LAT_P // NS, D // NC), F32)
    zeros_grid = jnp.zeros((GRID_P // NS, D // NC), F32)

    enc_src = _pad_idx(G["enc_src"], EP_E, 0, N_GRID)
    enc_dst = _pad_idx(G["enc_dst"], EP_E, N_LAT, LAT_P - N_LAT)
    proc_srcdst = jnp.concatenate([
        _pad_idx(G["proc_src"], EP_P, 0, N_LAT),
        _pad_idx(G["proc_dst"], EP_P, 0, N_LAT)])
    proc_dst = _pad_idx(G["proc_dst"], EP_P, N_LAT, LAT_P - N_LAT)
    dec_srcdst = jnp.concatenate([
        _pad_idx(G["dec_src"], EP_E, 0, N_LAT),
        LAT_P + _pad_idx(G["dec_dst"], EP_E, 0, N_GRID)])
    dec_dst_s = _pad_idx(G["dec_dst"], EP_E, N_GRID, GRID_P - N_GRID)

    pne = P["node_encoder"]
    x_grid, x_grid_pk = _mlp_ln(fpad, _pad2(pne["w1"], 128, D),
                                _vecs(pne["b1"], pne["b2"], pne["ln_g"],
                                      pne["ln_b"]),
                                pne["w2"], T_GRID, want_packed=True)

    def enc_attr(p, attr, n_pad):
        return _mlp_ln(_pad2(attr, n_pad, 128), _pad2(p["w1"], 128, D),
                       _vecs(p["b1"], p["b2"], p["ln_g"], p["ln_b"]),
                       p["w2"], T_EDGE)

    e_enc = enc_attr(P["enc_edge_encoder"], G["enc_attr"], EP_E)
    ep = enc_attr(P["proc_edge_encoder"], G["proc_attr"], EP_P)
    ed = enc_attr(P["dec_edge_encoder"], G["dec_attr"], EP_E)

    def edge_w(p):
        w1 = p["w1"].astype(jnp.bfloat16)
        return (w1[:D], w1[D:2 * D], w1[2 * D:],
                _vecs(p["b1"], p["b2"], p["ln_g"], p["ln_b"]),
                p["w2"].astype(jnp.bfloat16))

    def node_w(p):
        w1 = p["w1"]
        return (w1[:D], w1[D:],
                _vecs(p["b1"], p["b2"], p["ln_g"], p["ln_b"]), p["w2"])

    w1s, _, w1e, vecs, w2 = edge_w(P["enc_block"]["edge"])
    xs = _sc_gather(x_grid_pk, enc_src, EP_E)
    m = _edge_mlp(xs, 0, None, 0, e_enc, w1s, None, w1e, vecs, w2, EP_E, False)
    agg = _sc_scatter(m, enc_dst, zeros_lat, EP_E, LAT_P)
    w1x, w1a, nvecs, nw2 = node_w(P["enc_block"]["node"])
    x_lat, x_lat_pk = _node_mlp(None, [agg], None, w1a, nvecs, nw2, LAT_P,
                                T_LAT, want_packed=True)
    x_grid, x_grid_pk = _node_mlp(x_grid, [], w1x, None, nvecs, nw2, N_GRID,
                                  T_GRID, want_packed=True)

    nb_p = EP_P // T_EDGE
    for bp in P["proc_blocks"]:
        w1s, w1d, w1e, vecs, w2 = edge_w(bp["edge"])
        rows = _sc_gather(x_lat_pk, proc_srcdst, 2 * EP_P)
        m, ep = _edge_mlp(rows, 0, rows, nb_p, ep, w1s, w1d, w1e, vecs, w2,
                          EP_P, True)
        agg = _sc_scatter(m, proc_dst, zeros_lat, EP_P, LAT_P)
        w1x, w1a, nvecs, nw2 = node_w(bp["node"])
        x_lat, x_lat_pk = _node_mlp(x_lat, [agg], w1x, w1a, nvecs, nw2, LAT_P,
                                    T_LAT, want_packed=True)

    w1s, w1d, w1e, vecs, w2 = edge_w(P["dec_block"]["edge"])
    table = jnp.concatenate([x_lat_pk, x_grid_pk])
    rows = _sc_gather(table, dec_srcdst, 2 * EP_E)
    nb_e = EP_E // T_EDGE
    m = _edge_mlp(rows, 0, rows, nb_e, ed, w1s, w1d, w1e, vecs, w2, EP_E, False)
    agg = _sc_scatter(m, dec_dst_s, zeros_grid, EP_E, GRID_P)
    w1x, w1a, nvecs, nw2 = node_w(P["dec_block"]["node"])
    x_grid = _node_mlp(x_grid, [agg], w1x, w1a, nvecs, nw2, N_GRID, T_GRID)

    pd = P["node_decoder"]
    out = _final_mlp(x_grid, fpad, pd["w1"],
                     _vecs(pd["b1"], jnp.pad(pd["b2"], (0, 128 - D_IN))),
                     _pad2(pd["w2"], 128, 128))
    return out[:, :D_IN][None]


# speedup vs baseline: 1.6626x; 1.2692x over previous
import functools

import jax
import jax.numpy as jnp
from jax import lax
from jax.experimental import pallas as pl
from jax.experimental.pallas import tpu as pltpu
from jax.experimental.pallas import tpu_sc as plsc

D = 256
NC, NS = 2, 16
NW = NC * NS
N_GRID, N_LAT, D_IN = 10000, 2562, 78
EP_P = 16384
EP_E = 30720
LAT_P = 2688
GRID_P = 10240
T_EDGE = 1024
T_LAT = 336
T_GRID = 1000
F32 = jnp.float32


def _vecs(*vs):
    w = max(v.shape[0] for v in vs)
    out = jnp.zeros((8, w), F32)
    for i, v in enumerate(vs):
        out = out.at[i, : v.shape[0]].set(v)
    return out


def _ln(h, g, b):
    mu = jnp.mean(h, axis=-1, keepdims=True)
    var = jnp.mean((h - mu) ** 2, axis=-1, keepdims=True)
    return (h - mu) * lax.rsqrt(var + 1e-5) * g + b


def _dot(a, b):
    return jnp.dot(a, b, preferred_element_type=F32)


def _pack_bf16(x):
    lo = lax.bitcast_convert_type(x[:, :D // 2].astype(jnp.bfloat16),
                                  jnp.uint16).astype(jnp.uint32)
    hi = lax.bitcast_convert_type(x[:, D // 2:].astype(jnp.bfloat16),
                                  jnp.uint16).astype(jnp.uint32)
    return lax.bitcast_convert_type(lo | (hi << 16), jnp.int32)


def _unpack_bf16(xi):
    xu = lax.bitcast_convert_type(xi, jnp.uint32)
    lo = lax.bitcast_convert_type((xu & 0xFFFF).astype(jnp.uint16), jnp.bfloat16)
    hi = lax.bitcast_convert_type((xu >> 16).astype(jnp.uint16), jnp.bfloat16)
    return lo, hi


_CP = pltpu.CompilerParams(dimension_semantics=("parallel",))


def _mlp_ln(x, w1, vecs, w2, T, want_packed=False):
    n, din = x.shape
    dh, do = w1.shape[1], w2.shape[1]

    def body(x_r, w1_r, w2_r, v_r, o_r, *po):
        h = jnp.maximum(_dot(x_r[...], w1_r[...]) + v_r[0:1, :dh], 0.0)
        h = _dot(h, w2_r[...]) + v_r[1:2, :do]
        y = _ln(h, v_r[2:3, :do], v_r[3:4, :do])
        o_r[...] = y
        if want_packed:
            po[0][...] = _pack_bf16(y)

    out_specs = [pl.BlockSpec((T, do), lambda i: (i, 0))]
    out_shape = [jax.ShapeDtypeStruct((n, do), F32)]
    if want_packed:
        out_specs.append(pl.BlockSpec((T, do // 2), lambda i: (i, 0)))
        out_shape.append(jax.ShapeDtypeStruct((n, do // 2), jnp.int32))
    r = pl.pallas_call(
        body,
        grid=(n // T,),
        in_specs=[
            pl.BlockSpec((T, din), lambda i: (i, 0)),
            pl.BlockSpec((din, dh), lambda i: (0, 0)),
            pl.BlockSpec((dh, do), lambda i: (0, 0)),
            pl.BlockSpec((8, vecs.shape[1]), lambda i: (0, 0)),
        ],
        out_specs=out_specs if want_packed else out_specs[0],
        out_shape=out_shape if want_packed else out_shape[0],
        compiler_params=_CP,
    )(x, w1, w2, vecs)
    return r


def _edge_mlp(xs_arr, xs_off, xd_arr, xd_off, e, w1s, w1d, w1e, vecs, w2,
              n_edges, want_enew):
    nb = n_edges // T_EDGE
    has_xd = xd_arr is not None

    def body(*refs):
        if has_xd:
            xs_r, xd_r, e_r, w1s_r, w1d_r, w1e_r, w2_r, v_r, *outs = refs
        else:
            xs_r, e_r, w1s_r, w1e_r, w2_r, v_r, *outs = refs
        bf = jnp.bfloat16
        hd = D // 2
        lo, hi = _unpack_bf16(xs_r[...])
        acc = _dot(lo, w1s_r[:hd]) + _dot(hi, w1s_r[hd:])
        if has_xd:
            lo, hi = _unpack_bf16(xd_r[...])
            acc = acc + _dot(lo, w1d_r[:hd]) + _dot(hi, w1d_r[hd:])
        acc = acc + _dot(e_r[...].astype(bf), w1e_r[...]) + v_r[0:1, :]
        h = jnp.maximum(acc, 0.0)
        h = _dot(h.astype(bf), w2_r[...]) + v_r[1:2, :]
        m = _ln(h, v_r[2:3, :], v_r[3:4, :])
        outs[0][...] = m
        if want_enew:
            outs[1][...] = e_r[...] + m

    in_specs = [pl.BlockSpec((T_EDGE, D // 2), lambda i, o=xs_off: (i + o, 0))]
    inputs = [xs_arr]
    if has_xd:
        in_specs.append(
            pl.BlockSpec((T_EDGE, D // 2), lambda i, o=xd_off: (i + o, 0)))
        inputs.append(xd_arr)
    in_specs.append(pl.BlockSpec((T_EDGE, D), lambda i: (i, 0)))
    inputs.append(e)
    wspec = pl.BlockSpec((D, D), lambda i: (0, 0))
    in_specs += [wspec, wspec] if not has_xd else [wspec, wspec, wspec]
    inputs += [w1s, w1e] if not has_xd else [w1s, w1d, w1e]
    in_specs += [wspec, pl.BlockSpec((8, D), lambda i: (0, 0))]
    inputs += [w2, vecs]

    out_spec = pl.BlockSpec((T_EDGE, D), lambda i: (i, 0))
    out_shape = jax.ShapeDtypeStruct((n_edges, D), F32)
    if want_enew:
        out_specs, out_shapes = [out_spec, out_spec], [out_shape, out_shape]
    else:
        out_specs, out_shapes = out_spec, out_shape

    return pl.pallas_call(
        body,
        grid=(nb,),
        in_specs=in_specs,
        out_specs=out_specs,
        out_shape=out_shapes,
        compiler_params=_CP,
    )(*inputs)


def _node_mlp(x, aggs, w1x, w1a, vecs, w2, n_rows, T, want_packed=False):
    use_x = x is not None

    def body(*refs):
        refs = list(refs)
        p_r = refs.pop() if want_packed else None
        o_r = refs.pop()
        v_r = refs.pop()
        w2_r = refs.pop()
        x_r = refs.pop(0) if use_x else None
        agg_rs = [refs.pop(0) for _ in aggs]
        w1x_r = refs.pop(0) if use_x else None
        w1a_r = refs.pop(0) if aggs else None
        acc = v_r[0:1, :] * jnp.ones((T, 1), F32)
        if use_x:
            acc = acc + _dot(x_r[...], w1x_r[...])
        if agg_rs:
            a = agg_rs[0][...]
            for r in agg_rs[1:]:
                a = a + r[...]
            acc = acc + _dot(a, w1a_r[...])
        h = jnp.maximum(acc, 0.0)
        h = _dot(h, w2_r[...]) + v_r[1:2, :]
        m = _ln(h, v_r[2:3, :], v_r[3:4, :])
        y = (x_r[...] + m) if use_x else m
        o_r[...] = y
        if want_packed:
            p_r[...] = _pack_bf16(y)

    rspec = pl.BlockSpec((T, D), lambda i: (i, 0))
    wspec = pl.BlockSpec((D, D), lambda i: (0, 0))
    inputs, in_specs = [], []
    if use_x:
        inputs.append(x); in_specs.append(rspec)
    for a in aggs:
        inputs.append(a); in_specs.append(rspec)
    if use_x:
        inputs.append(w1x); in_specs.append(wspec)
    if aggs:
        inputs.append(w1a); in_specs.append(wspec)
    inputs += [w2, vecs]
    in_specs += [wspec, pl.BlockSpec((8, D), lambda i: (0, 0))]

    out_specs = [rspec]
    out_shape = [jax.ShapeDtypeStruct((n_rows, D), F32)]
    if want_packed:
        out_specs.append(pl.BlockSpec((T, D // 2), lambda i: (i, 0)))
        out_shape.append(jax.ShapeDtypeStruct((n_rows, D // 2), jnp.int32))
    return pl.pallas_call(
        body,
        grid=(n_rows // T,),
        in_specs=in_specs,
        out_specs=out_specs if want_packed else out_specs[0],
        out_shape=out_shape if want_packed else out_shape[0],
        compiler_params=_CP,
    )(*inputs)


def _final_mlp(x, featp, w1, vecs, w2):
    n = x.shape[0]
    dh = w1.shape[1]

    def body(x_r, f_r, w1_r, w2_r, v_r, o_r):
        h = jnp.maximum(_dot(x_r[...], w1_r[...]) + v_r[0:1, :], 0.0)
        o_r[...] = _dot(h, w2_r[...]) + v_r[1:2, :] + f_r[...]

    return pl.pallas_call(
        body,
        grid=(n // T_GRID,),
        in_specs=[
            pl.BlockSpec((T_GRID, D), lambda i: (i, 0)),
            pl.BlockSpec((T_GRID, dh), lambda i: (i, 0)),
            pl.BlockSpec((D, dh), lambda i: (0, 0)),
            pl.BlockSpec((dh, dh), lambda i: (0, 0)),
            pl.BlockSpec((8, dh), lambda i: (0, 0)),
        ],
        out_specs=pl.BlockSpec((T_GRID, dh), lambda i: (i, 0)),
        out_shape=jax.ShapeDtypeStruct((n, dh), F32),
        compiler_params=_CP,
    )(x, featp, w1, w2, vecs)


@functools.cache
def _mesh():
    return plsc.VectorSubcoreMesh(core_axis_name="c", subcore_axis_name="s",
                                  num_cores=NC, num_subcores=NS)


_CHUNK = 128


def _sc_gather(table, idx, n_out):
    return _sc_gather_raw(table, idx, n_out, table.shape[1])


def _sc_gather_raw(table, idx, n_out, w):
    bpw = n_out // NW
    nfull, rem = divmod(bpw, _CHUNK)
    cps = [_CHUNK] * nfull + ([rem] if rem else [])
    nch = len(cps)
    dt = table.dtype
    scratch = [pltpu.VMEM((bpw,), jnp.int32),
               pltpu.VMEM((_CHUNK, w), dt), pltpu.VMEM((_CHUNK, w), dt),
               pltpu.SemaphoreType.DMA, pltpu.SemaphoreType.DMA,
               pltpu.SemaphoreType.DMA, pltpu.SemaphoreType.DMA]

    @functools.partial(
        pl.kernel,
        out_type=jax.ShapeDtypeStruct((n_out, w), dt),
        mesh=_mesh(),
        scratch_types=scratch,
    )
    def k(table_hbm, idx_hbm, out_hbm, idx_v, r0, r1, sg0, sg1, sw0, sw1):
        wid = lax.axis_index("s") * NC + lax.axis_index("c")
        base = wid * bpw
        pltpu.sync_copy(idx_hbm.at[pl.ds(base, bpw)], idx_v)
        rows, sgs, sws = (r0, r1), (sg0, sg1), (sw0, sw1)

        def buf(i):
            b = i & 1
            sz = cps[i]
            return rows[b] if sz == _CHUNK else rows[b].at[pl.ds(0, sz)]

        def issue_g(i):
            sz = cps[i]
            return pltpu.async_copy(
                table_hbm.at[idx_v.at[pl.ds(i * _CHUNK, sz)]], buf(i),
                sgs[i & 1])

        def issue_w(i):
            sz = cps[i]
            return pltpu.async_copy(
                buf(i), out_hbm.at[pl.ds(base + i * _CHUNK, sz)], sws[i & 1])

        gd, wd = [None] * nch, [None] * nch
        gd[0] = issue_g(0)
        for i in range(nch):
            if i + 1 < nch:
                if i >= 1:
                    wd[i - 1].wait()
                gd[i + 1] = issue_g(i + 1)
            gd[i].wait()
            wd[i] = issue_w(i)
        wd[nch - 1].wait()
        if nch >= 2:
            wd[nch - 2].wait()

    return k(table, idx)


def _sc_scatter(m, dst, zeros, n_edges, n_nodes):
    dh = D // NC
    epw = n_edges // NS
    nch = epw // _CHUNK
    assert nch * _CHUNK == epw
    rps = n_nodes // NS
    scratch = [pltpu.VMEM((1, _CHUNK), jnp.int32), pltpu.VMEM((1, _CHUNK), jnp.int32),
               pltpu.VMEM((_CHUNK, dh), F32), pltpu.VMEM((_CHUNK, dh), F32),
               pltpu.SemaphoreType.DMA, pltpu.SemaphoreType.DMA,
               pltpu.SemaphoreType.DMA, pltpu.SemaphoreType.DMA,
               pltpu.SemaphoreType.DMA, pltpu.SemaphoreType.DMA,
               pltpu.VMEM_SHARED((n_nodes, dh), F32)]

    @functools.partial(
        pl.kernel,
        out_type=jax.ShapeDtypeStruct((n_nodes, D), F32),
        mesh=_mesh(),
        scratch_types=scratch,
    )
    def k(m_hbm, dst_hbm, z_hbm, out_hbm, i0, i1, m0, m1,
          si0, si1, sm0, sm1, ss0, ss1, acc):
        cid = lax.axis_index("c")
        sid = lax.axis_index("s")
        col = cid * dh
        pltpu.sync_copy(z_hbm, acc.at[pl.ds(sid * rps, rps)])
        plsc.subcore_barrier()
        base = sid * epw
        idxs, mbufs = (i0, i1), (m0, m1)
        sis, sms, sss = (si0, si1), (sm0, sm1), (ss0, ss1)

        def issue_l(i):
            b = i & 1
            off = base + i * _CHUNK
            di = pltpu.async_copy(dst_hbm.at[pl.ds(off, _CHUNK)],
                                  idxs[b].at[0], sis[b])
            dm = pltpu.async_copy(m_hbm.at[pl.ds(off, _CHUNK), pl.ds(col, dh)],
                                  mbufs[b], sms[b])
            return di, dm

        def issue_s(i):
            b = i & 1
            return pltpu.async_copy(mbufs[b], acc.at[idxs[b].at[0]], sss[b],
                                    add=True)

        ld, sd = [None] * nch, [None] * nch
        ld[0] = issue_l(0)
        for i in range(nch):
            if i + 1 < nch:
                if i >= 1:
                    sd[i - 1].wait()
                ld[i + 1] = issue_l(i + 1)
            ld[i][0].wait()
            ld[i][1].wait()
            sd[i] = issue_s(i)
        sd[nch - 1].wait()
        if nch >= 2:
            sd[nch - 2].wait()
        plsc.subcore_barrier()
        pltpu.sync_copy(acc.at[pl.ds(sid * rps, rps)],
                        out_hbm.at[pl.ds(sid * rps, rps), pl.ds(col, dh)])

    return k(m, dst, zeros)


def _pad_idx(a, n, fill_start, fill_mod):
    npad = n - a.shape[0]
    pad = fill_start + (jnp.arange(npad, dtype=jnp.int32) % fill_mod)
    return jnp.concatenate([a.astype(jnp.int32), pad])


def _pad2(a, rows, cols):
    return jnp.pad(a, ((0, rows - a.shape[0]), (0, cols - a.shape[1])))


def kernel(features, params, graph):
    f = features[0]
    P, G = params, graph

    fpad = _pad2(f, N_GRID, 128)
    zeros_lat = jnp.zeros((LAT_P // NS, D // NC), F32)
    zeros_grid = jnp.zeros((GRID_P // NS, D // NC), F32)

    enc_src = _pad_idx(G["enc_src"], EP_E, 0, N_GRID)
    enc_dst = _pad_idx(G["enc_dst"], EP_E, N_LAT, LAT_P - N_LAT)
    proc_srcdst = jnp.concatenate([
        _pad_idx(G["proc_src"], EP_P, 0, N_LAT),
        _pad_idx(G["proc_dst"], EP_P, 0, N_LAT)])
    proc_dst = _pad_idx(G["proc_dst"], EP_P, N_LAT, LAT_P - N_LAT)
    dec_srcdst = jnp.concatenate([
        _pad_idx(G["dec_src"], EP_E, 0, N_LAT),
        LAT_P + _pad_idx(G["dec_dst"], EP_E, 0, N_GRID)])
    dec_dst_s = _pad_idx(G["dec_dst"], EP_E, N_GRID, GRID_P - N_GRID)

    pne = P["node_encoder"]
    x_grid, x_grid_pk = _mlp_ln(fpad, _pad2(pne["w1"], 128, D),
                                _vecs(pne["b1"], pne["b2"], pne["ln_g"],
                                      pne["ln_b"]),
                                pne["w2"], T_GRID, want_packed=True)

    def enc_attr(p, attr, n_pad):
        return _mlp_ln(_pad2(attr, n_pad, 128), _pad2(p["w1"], 128, D),
                       _vecs(p["b1"], p["b2"], p["ln_g"], p["ln_b"]),
                       p["w2"], T_EDGE)

    e_enc = enc_attr(P["enc_edge_encoder"], G["enc_attr"], EP_E)
    ep = enc_attr(P["proc_edge_encoder"], G["proc_attr"], EP_P)
    ed = enc_attr(P["dec_edge_encoder"], G["dec_attr"], EP_E)

    def edge_w(p):
        w1 = p["w1"].astype(jnp.bfloat16)
        return (w1[:D], w1[D:2 * D], w1[2 * D:],
                _vecs(p["b1"], p["b2"], p["ln_g"], p["ln_b"]),
                p["w2"].astype(jnp.bfloat16))

    def node_w(p):
        w1 = p["w1"]
        return (w1[:D], w1[D:],
                _vecs(p["b1"], p["b2"], p["ln_g"], p["ln_b"]), p["w2"])

    w1s, _, w1e, vecs, w2 = edge_w(P["enc_block"]["edge"])
    xs = _sc_gather(x_grid_pk, enc_src, EP_E)
    m = _edge_mlp(xs, 0, None, 0, e_enc, w1s, None, w1e, vecs, w2, EP_E, False)
    agg = _sc_scatter(m, enc_dst, zeros_lat, EP_E, LAT_P)
    w1x, w1a, nvecs, nw2 = node_w(P["enc_block"]["node"])
    x_lat, x_lat_pk = _node_mlp(None, [agg], None, w1a, nvecs, nw2, LAT_P,
                                T_LAT, want_packed=True)
    x_grid, x_grid_pk = _node_mlp(x_grid, [], w1x, None, nvecs, nw2, N_GRID,
                                  T_GRID, want_packed=True)

    nb_p = EP_P // T_EDGE
    for bp in P["proc_blocks"]:
        w1s, w1d, w1e, vecs, w2 = edge_w(bp["edge"])
        rows = _sc_gather(x_lat_pk, proc_srcdst, 2 * EP_P)
        m, ep = _edge_mlp(rows, 0, rows, nb_p, ep, w1s, w1d, w1e, vecs, w2,
                          EP_P, True)
        agg = _sc_scatter(m, proc_dst, zeros_lat, EP_P, LAT_P)
        w1x, w1a, nvecs, nw2 = node_w(bp["node"])
        x_lat, x_lat_pk = _node_mlp(x_lat, [agg], w1x, w1a, nvecs, nw2, LAT_P,
                                    T_LAT, want_packed=True)

    w1s, w1d, w1e, vecs, w2 = edge_w(P["dec_block"]["edge"])
    table = jnp.concatenate([x_lat_pk, x_grid_pk])
    rows = _sc_gather(table, dec_srcdst, 2 * EP_E)
    nb_e = EP_E // T_EDGE
    m = _edge_mlp(rows, 0, rows, nb_e, ed, w1s, w1d, w1e, vecs, w2, EP_E, False)
    agg = _sc_scatter(m, dec_dst_s, zeros_grid, EP_E, GRID_P)
    w1x, w1a, nvecs, nw2 = node_w(P["dec_block"]["node"])
    x_grid = _node_mlp(x_grid, [agg], w1x, w1a, nvecs, nw2, N_GRID, T_GRID)

    pd = P["node_decoder"]
    out = _final_mlp(x_grid, fpad, pd["w1"],
                     _vecs(pd["b1"], jnp.pad(pd["b2"], (0, 128 - D_IN))),
                     _pad2(pd["w2"], 128, 128))
    return out[:, :D_IN][None]


# speedup vs baseline: 1.8488x; 1.1120x over previous
import functools

import jax
import jax.numpy as jnp
from jax import lax
from jax.experimental import pallas as pl
from jax.experimental.pallas import tpu as pltpu
from jax.experimental.pallas import tpu_sc as plsc

D = 256
NC, NS = 2, 16
NW = NC * NS
N_GRID, N_LAT, D_IN = 10000, 2562, 78
EP_P = 16384
EP_E = 30720
LAT_P = 2688
GRID_P = 10240
T_EDGE = 2048
T_LAT = 672
T_GRID = 2000
F32 = jnp.float32


def _vecs(*vs):
    w = max(v.shape[0] for v in vs)
    out = jnp.zeros((8, w), F32)
    for i, v in enumerate(vs):
        out = out.at[i, : v.shape[0]].set(v)
    return out


def _ln(h, g, b):
    mu = jnp.mean(h, axis=-1, keepdims=True)
    var = jnp.mean((h - mu) ** 2, axis=-1, keepdims=True)
    return (h - mu) * lax.rsqrt(var + 1e-5) * g + b


def _dot(a, b):
    return jnp.dot(a, b, preferred_element_type=F32)


def _pack_bf16(x):
    lo = lax.bitcast_convert_type(x[:, :D // 2].astype(jnp.bfloat16),
                                  jnp.uint16).astype(jnp.uint32)
    hi = lax.bitcast_convert_type(x[:, D // 2:].astype(jnp.bfloat16),
                                  jnp.uint16).astype(jnp.uint32)
    return lax.bitcast_convert_type(lo | (hi << 16), jnp.int32)


def _unpack_bf16(xi):
    xu = lax.bitcast_convert_type(xi, jnp.uint32)
    lo = lax.bitcast_convert_type((xu & 0xFFFF).astype(jnp.uint16), jnp.bfloat16)
    hi = lax.bitcast_convert_type((xu >> 16).astype(jnp.uint16), jnp.bfloat16)
    return lo, hi


_CP = pltpu.CompilerParams(dimension_semantics=("parallel",))


def _mlp_ln(x, w1, vecs, w2, T, want_packed=False):
    n, din = x.shape
    dh, do = w1.shape[1], w2.shape[1]

    def body(x_r, w1_r, w2_r, v_r, o_r, *po):
        h = jnp.maximum(_dot(x_r[...], w1_r[...]) + v_r[0:1, :dh], 0.0)
        h = _dot(h, w2_r[...]) + v_r[1:2, :do]
        y = _ln(h, v_r[2:3, :do], v_r[3:4, :do])
        o_r[...] = y
        if want_packed:
            po[0][...] = _pack_bf16(y)

    out_specs = [pl.BlockSpec((T, do), lambda i: (i, 0))]
    out_shape = [jax.ShapeDtypeStruct((n, do), F32)]
    if want_packed:
        out_specs.append(pl.BlockSpec((T, do // 2), lambda i: (i, 0)))
        out_shape.append(jax.ShapeDtypeStruct((n, do // 2), jnp.int32))
    r = pl.pallas_call(
        body,
        grid=(n // T,),
        in_specs=[
            pl.BlockSpec((T, din), lambda i: (i, 0)),
            pl.BlockSpec((din, dh), lambda i: (0, 0)),
            pl.BlockSpec((dh, do), lambda i: (0, 0)),
            pl.BlockSpec((8, vecs.shape[1]), lambda i: (0, 0)),
        ],
        out_specs=out_specs if want_packed else out_specs[0],
        out_shape=out_shape if want_packed else out_shape[0],
        compiler_params=_CP,
    )(x, w1, w2, vecs)
    return r


def _edge_mlp(xs_arr, xs_off, xd_arr, xd_off, e, w1s, w1d, w1e, vecs, w2,
              n_edges, want_enew):
    nb = n_edges // T_EDGE
    has_xd = xd_arr is not None

    def body(*refs):
        if has_xd:
            xs_r, xd_r, e_r, w1s_r, w1d_r, w1e_r, w2_r, v_r, *outs = refs
        else:
            xs_r, e_r, w1s_r, w1e_r, w2_r, v_r, *outs = refs
        bf = jnp.bfloat16
        hd = D // 2
        lo, hi = _unpack_bf16(xs_r[...])
        acc = _dot(lo, w1s_r[:hd]) + _dot(hi, w1s_r[hd:])
        if has_xd:
            lo, hi = _unpack_bf16(xd_r[...])
            acc = acc + _dot(lo, w1d_r[:hd]) + _dot(hi, w1d_r[hd:])
        acc = acc + _dot(e_r[...].astype(bf), w1e_r[...]) + v_r[0:1, :]
        h = jnp.maximum(acc, 0.0)
        h = _dot(h.astype(bf), w2_r[...]) + v_r[1:2, :]
        m = _ln(h, v_r[2:3, :], v_r[3:4, :])
        outs[0][...] = m
        if want_enew:
            outs[1][...] = e_r[...] + m

    in_specs = [pl.BlockSpec((T_EDGE, D // 2), lambda i, o=xs_off: (i + o, 0))]
    inputs = [xs_arr]
    if has_xd:
        in_specs.append(
            pl.BlockSpec((T_EDGE, D // 2), lambda i, o=xd_off: (i + o, 0)))
        inputs.append(xd_arr)
    in_specs.append(pl.BlockSpec((T_EDGE, D), lambda i: (i, 0)))
    inputs.append(e)
    wspec = pl.BlockSpec((D, D), lambda i: (0, 0))
    in_specs += [wspec, wspec] if not has_xd else [wspec, wspec, wspec]
    inputs += [w1s, w1e] if not has_xd else [w1s, w1d, w1e]
    in_specs += [wspec, pl.BlockSpec((8, D), lambda i: (0, 0))]
    inputs += [w2, vecs]

    out_spec = pl.BlockSpec((T_EDGE, D), lambda i: (i, 0))
    out_shape = jax.ShapeDtypeStruct((n_edges, D), F32)
    if want_enew:
        out_specs, out_shapes = [out_spec, out_spec], [out_shape, out_shape]
    else:
        out_specs, out_shapes = out_spec, out_shape

    return pl.pallas_call(
        body,
        grid=(nb,),
        in_specs=in_specs,
        out_specs=out_specs,
        out_shape=out_shapes,
        compiler_params=_CP,
    )(*inputs)


def _node_mlp(x, aggs, w1x, w1a, vecs, w2, n_rows, T, want_packed=False):
    use_x = x is not None

    def body(*refs):
        refs = list(refs)
        p_r = refs.pop() if want_packed else None
        o_r = refs.pop()
        v_r = refs.pop()
        w2_r = refs.pop()
        x_r = refs.pop(0) if use_x else None
        agg_rs = [refs.pop(0) for _ in aggs]
        w1x_r = refs.pop(0) if use_x else None
        w1a_r = refs.pop(0) if aggs else None
        acc = v_r[0:1, :] * jnp.ones((T, 1), F32)
        if use_x:
            acc = acc + _dot(x_r[...], w1x_r[...])
        if agg_rs:
            a = agg_rs[0][...]
            for r in agg_rs[1:]:
                a = a + r[...]
            acc = acc + _dot(a, w1a_r[...])
        h = jnp.maximum(acc, 0.0)
        h = _dot(h, w2_r[...]) + v_r[1:2, :]
        m = _ln(h, v_r[2:3, :], v_r[3:4, :])
        y = (x_r[...] + m) if use_x else m
        o_r[...] = y
        if want_packed:
            p_r[...] = _pack_bf16(y)

    rspec = pl.BlockSpec((T, D), lambda i: (i, 0))
    wspec = pl.BlockSpec((D, D), lambda i: (0, 0))
    inputs, in_specs = [], []
    if use_x:
        inputs.append(x); in_specs.append(rspec)
    for a in aggs:
        inputs.append(a); in_specs.append(rspec)
    if use_x:
        inputs.append(w1x); in_specs.append(wspec)
    if aggs:
        inputs.append(w1a); in_specs.append(wspec)
    inputs += [w2, vecs]
    in_specs += [wspec, pl.BlockSpec((8, D), lambda i: (0, 0))]

    out_specs = [rspec]
    out_shape = [jax.ShapeDtypeStruct((n_rows, D), F32)]
    if want_packed:
        out_specs.append(pl.BlockSpec((T, D // 2), lambda i: (i, 0)))
        out_shape.append(jax.ShapeDtypeStruct((n_rows, D // 2), jnp.int32))
    return pl.pallas_call(
        body,
        grid=(n_rows // T,),
        in_specs=in_specs,
        out_specs=out_specs if want_packed else out_specs[0],
        out_shape=out_shape if want_packed else out_shape[0],
        compiler_params=_CP,
    )(*inputs)


def _final_mlp(x, featp, w1, vecs, w2):
    n = x.shape[0]
    dh = w1.shape[1]

    def body(x_r, f_r, w1_r, w2_r, v_r, o_r):
        h = jnp.maximum(_dot(x_r[...], w1_r[...]) + v_r[0:1, :], 0.0)
        o_r[...] = _dot(h, w2_r[...]) + v_r[1:2, :] + f_r[...]

    return pl.pallas_call(
        body,
        grid=(n // T_GRID,),
        in_specs=[
            pl.BlockSpec((T_GRID, D), lambda i: (i, 0)),
            pl.BlockSpec((T_GRID, dh), lambda i: (i, 0)),
            pl.BlockSpec((D, dh), lambda i: (0, 0)),
            pl.BlockSpec((dh, dh), lambda i: (0, 0)),
            pl.BlockSpec((8, dh), lambda i: (0, 0)),
        ],
        out_specs=pl.BlockSpec((T_GRID, dh), lambda i: (i, 0)),
        out_shape=jax.ShapeDtypeStruct((n, dh), F32),
        compiler_params=_CP,
    )(x, featp, w1, w2, vecs)


@functools.cache
def _mesh():
    return plsc.VectorSubcoreMesh(core_axis_name="c", subcore_axis_name="s",
                                  num_cores=NC, num_subcores=NS)


_CHUNK = 128


def _sc_gather(table, idx, n_out):
    return _sc_gather_raw(table, idx, n_out, table.shape[1])


def _sc_gather_raw(table, idx, n_out, w):
    bpw = n_out // NW
    nfull, rem = divmod(bpw, _CHUNK)
    cps = [_CHUNK] * nfull + ([rem] if rem else [])
    nch = len(cps)
    dt = table.dtype
    scratch = [pltpu.VMEM((bpw,), jnp.int32),
               pltpu.VMEM((_CHUNK, w), dt), pltpu.VMEM((_CHUNK, w), dt),
               pltpu.SemaphoreType.DMA, pltpu.SemaphoreType.DMA,
               pltpu.SemaphoreType.DMA, pltpu.SemaphoreType.DMA]

    @functools.partial(
        pl.kernel,
        out_type=jax.ShapeDtypeStruct((n_out, w), dt),
        mesh=_mesh(),
        scratch_types=scratch,
    )
    def k(table_hbm, idx_hbm, out_hbm, idx_v, r0, r1, sg0, sg1, sw0, sw1):
        wid = lax.axis_index("s") * NC + lax.axis_index("c")
        base = wid * bpw
        pltpu.sync_copy(idx_hbm.at[pl.ds(base, bpw)], idx_v)
        rows, sgs, sws = (r0, r1), (sg0, sg1), (sw0, sw1)

        def buf(i):
            b = i & 1
            sz = cps[i]
            return rows[b] if sz == _CHUNK else rows[b].at[pl.ds(0, sz)]

        def issue_g(i):
            sz = cps[i]
            return pltpu.async_copy(
                table_hbm.at[idx_v.at[pl.ds(i * _CHUNK, sz)]], buf(i),
                sgs[i & 1])

        def issue_w(i):
            sz = cps[i]
            return pltpu.async_copy(
                buf(i), out_hbm.at[pl.ds(base + i * _CHUNK, sz)], sws[i & 1])

        gd, wd = [None] * nch, [None] * nch
        gd[0] = issue_g(0)
        for i in range(nch):
            if i + 1 < nch:
                if i >= 1:
                    wd[i - 1].wait()
                gd[i + 1] = issue_g(i + 1)
            gd[i].wait()
            wd[i] = issue_w(i)
        wd[nch - 1].wait()
        if nch >= 2:
            wd[nch - 2].wait()

    return k(table, idx)


def _sc_scatter(m, dst, zeros, n_edges, n_nodes):
    dh = D // NC
    epw = n_edges // NS
    nch = epw // _CHUNK
    assert nch * _CHUNK == epw
    rps = n_nodes // NS
    scratch = [pltpu.VMEM((1, _CHUNK), jnp.int32), pltpu.VMEM((1, _CHUNK), jnp.int32),
               pltpu.VMEM((_CHUNK, dh), F32), pltpu.VMEM((_CHUNK, dh), F32),
               pltpu.SemaphoreType.DMA, pltpu.SemaphoreType.DMA,
               pltpu.SemaphoreType.DMA, pltpu.SemaphoreType.DMA,
               pltpu.SemaphoreType.DMA, pltpu.SemaphoreType.DMA,
               pltpu.VMEM_SHARED((n_nodes, dh), F32)]

    @functools.partial(
        pl.kernel,
        out_type=jax.ShapeDtypeStruct((n_nodes, D), F32),
        mesh=_mesh(),
        scratch_types=scratch,
    )
    def k(m_hbm, dst_hbm, z_hbm, out_hbm, i0, i1, m0, m1,
          si0, si1, sm0, sm1, ss0, ss1, acc):
        cid = lax.axis_index("c")
        sid = lax.axis_index("s")
        col = cid * dh
        pltpu.sync_copy(z_hbm, acc.at[pl.ds(sid * rps, rps)])
        plsc.subcore_barrier()
        base = sid * epw
        idxs, mbufs = (i0, i1), (m0, m1)
        sis, sms, sss = (si0, si1), (sm0, sm1), (ss0, ss1)

        def issue_l(i):
            b = i & 1
            off = base + i * _CHUNK
            di = pltpu.async_copy(dst_hbm.at[pl.ds(off, _CHUNK)],
                                  idxs[b].at[0], sis[b])
            dm = pltpu.async_copy(m_hbm.at[pl.ds(off, _CHUNK), pl.ds(col, dh)],
                                  mbufs[b], sms[b])
            return di, dm

        def issue_s(i):
            b = i & 1
            return pltpu.async_copy(mbufs[b], acc.at[idxs[b].at[0]], sss[b],
                                    add=True)

        ld, sd = [None] * nch, [None] * nch
        ld[0] = issue_l(0)
        for i in range(nch):
            if i + 1 < nch:
                if i >= 1:
                    sd[i - 1].wait()
                ld[i + 1] = issue_l(i + 1)
            ld[i][0].wait()
            ld[i][1].wait()
            sd[i] = issue_s(i)
        sd[nch - 1].wait()
        if nch >= 2:
            sd[nch - 2].wait()
        plsc.subcore_barrier()
        pltpu.sync_copy(acc.at[pl.ds(sid * rps, rps)],
                        out_hbm.at[pl.ds(sid * rps, rps), pl.ds(col, dh)])

    return k(m, dst, zeros)


def _pad_idx(a, n, fill_start, fill_mod):
    npad = n - a.shape[0]
    pad = fill_start + (jnp.arange(npad, dtype=jnp.int32) % fill_mod)
    return jnp.concatenate([a.astype(jnp.int32), pad])


def _pad2(a, rows, cols):
    return jnp.pad(a, ((0, rows - a.shape[0]), (0, cols - a.shape[1])))


def kernel(features, params, graph):
    f = features[0]
    P, G = params, graph

    fpad = _pad2(f, N_GRID, 128)
    zeros_lat = jnp.zeros((LAT_P // NS, D // NC), F32)
    zeros_grid = jnp.zeros((GRID_P // NS, D // NC), F32)

    enc_src = _pad_idx(G["enc_src"], EP_E, 0, N_GRID)
    enc_dst = _pad_idx(G["enc_dst"], EP_E, N_LAT, LAT_P - N_LAT)
    proc_srcdst = jnp.concatenate([
        _pad_idx(G["proc_src"], EP_P, 0, N_LAT),
        _pad_idx(G["proc_dst"], EP_P, 0, N_LAT)])
    proc_dst = _pad_idx(G["proc_dst"], EP_P, N_LAT, LAT_P - N_LAT)
    dec_srcdst = jnp.concatenate([
        _pad_idx(G["dec_src"], EP_E, 0, N_LAT),
        LAT_P + _pad_idx(G["dec_dst"], EP_E, 0, N_GRID)])
    dec_dst_s = _pad_idx(G["dec_dst"], EP_E, N_GRID, GRID_P - N_GRID)

    pne = P["node_encoder"]
    x_grid, x_grid_pk = _mlp_ln(fpad, _pad2(pne["w1"], 128, D),
                                _vecs(pne["b1"], pne["b2"], pne["ln_g"],
                                      pne["ln_b"]),
                                pne["w2"], T_GRID, want_packed=True)

    def enc_attr(p, attr, n_pad):
        return _mlp_ln(_pad2(attr, n_pad, 128), _pad2(p["w1"], 128, D),
                       _vecs(p["b1"], p["b2"], p["ln_g"], p["ln_b"]),
                       p["w2"], T_EDGE)

    e_enc = enc_attr(P["enc_edge_encoder"], G["enc_attr"], EP_E)
    ep = enc_attr(P["proc_edge_encoder"], G["proc_attr"], EP_P)
    ed = enc_attr(P["dec_edge_encoder"], G["dec_attr"], EP_E)

    def edge_w(p):
        w1 = p["w1"].astype(jnp.bfloat16)
        return (w1[:D], w1[D:2 * D], w1[2 * D:],
                _vecs(p["b1"], p["b2"], p["ln_g"], p["ln_b"]),
                p["w2"].astype(jnp.bfloat16))

    def node_w(p):
        w1 = p["w1"]
        return (w1[:D], w1[D:],
                _vecs(p["b1"], p["b2"], p["ln_g"], p["ln_b"]), p["w2"])

    w1s, _, w1e, vecs, w2 = edge_w(P["enc_block"]["edge"])
    xs = _sc_gather(x_grid_pk, enc_src, EP_E)
    m = _edge_mlp(xs, 0, None, 0, e_enc, w1s, None, w1e, vecs, w2, EP_E, False)
    agg = _sc_scatter(m, enc_dst, zeros_lat, EP_E, LAT_P)
    w1x, w1a, nvecs, nw2 = node_w(P["enc_block"]["node"])
    x_lat, x_lat_pk = _node_mlp(None, [agg], None, w1a, nvecs, nw2, LAT_P,
                                T_LAT, want_packed=True)
    x_grid, x_grid_pk = _node_mlp(x_grid, [], w1x, None, nvecs, nw2, N_GRID,
                                  T_GRID, want_packed=True)

    nb_p = EP_P // T_EDGE
    for bp in P["proc_blocks"]:
        w1s, w1d, w1e, vecs, w2 = edge_w(bp["edge"])
        rows = _sc_gather(x_lat_pk, proc_srcdst, 2 * EP_P)
        m, ep = _edge_mlp(rows, 0, rows, nb_p, ep, w1s, w1d, w1e, vecs, w2,
                          EP_P, True)
        agg = _sc_scatter(m, proc_dst, zeros_lat, EP_P, LAT_P)
        w1x, w1a, nvecs, nw2 = node_w(bp["node"])
        x_lat, x_lat_pk = _node_mlp(x_lat, [agg], w1x, w1a, nvecs, nw2, LAT_P,
                                    T_LAT, want_packed=True)

    w1s, w1d, w1e, vecs, w2 = edge_w(P["dec_block"]["edge"])
    table = jnp.concatenate([x_lat_pk, x_grid_pk])
    rows = _sc_gather(table, dec_srcdst, 2 * EP_E)
    nb_e = EP_E // T_EDGE
    m = _edge_mlp(rows, 0, rows, nb_e, ed, w1s, w1d, w1e, vecs, w2, EP_E, False)
    agg = _sc_scatter(m, dec_dst_s, zeros_grid, EP_E, GRID_P)
    w1x, w1a, nvecs, nw2 = node_w(P["dec_block"]["node"])
    x_grid = _node_mlp(x_grid, [agg], w1x, w1a, nvecs, nw2, N_GRID, T_GRID)

    pd = P["node_decoder"]
    out = _final_mlp(x_grid, fpad, pd["w1"],
                     _vecs(pd["b1"], jnp.pad(pd["b2"], (0, 128 - D_IN))),
                     _pad2(pd["w2"], 128, 128))
    return out[:, :D_IN][None]


# speedup vs baseline: 1.8543x; 1.0030x over previous
import functools

import jax
import jax.numpy as jnp
from jax import lax
from jax.experimental import pallas as pl
from jax.experimental.pallas import tpu as pltpu
from jax.experimental.pallas import tpu_sc as plsc

D = 256
NC, NS = 2, 16
NW = NC * NS
N_GRID, N_LAT, D_IN = 10000, 2562, 78
EP_P = 16384
EP_E = 30720
LAT_P = 2688
GRID_P = 10240
T_EP = 4096
T_EE = 2048
T_LAT = 672
T_GRID = 2000
F32 = jnp.float32


def _vecs(*vs):
    w = max(v.shape[0] for v in vs)
    out = jnp.zeros((8, w), F32)
    for i, v in enumerate(vs):
        out = out.at[i, : v.shape[0]].set(v)
    return out


def _ln(h, g, b):
    mu = jnp.mean(h, axis=-1, keepdims=True)
    var = jnp.mean((h - mu) ** 2, axis=-1, keepdims=True)
    return (h - mu) * lax.rsqrt(var + 1e-5) * g + b


def _dot(a, b):
    return jnp.dot(a, b, preferred_element_type=F32)


def _pack_bf16(x):
    lo = lax.bitcast_convert_type(x[:, :D // 2].astype(jnp.bfloat16),
                                  jnp.uint16).astype(jnp.uint32)
    hi = lax.bitcast_convert_type(x[:, D // 2:].astype(jnp.bfloat16),
                                  jnp.uint16).astype(jnp.uint32)
    return lax.bitcast_convert_type(lo | (hi << 16), jnp.int32)


def _unpack_bf16(xi):
    xu = lax.bitcast_convert_type(xi, jnp.uint32)
    lo = lax.bitcast_convert_type((xu & 0xFFFF).astype(jnp.uint16), jnp.bfloat16)
    hi = lax.bitcast_convert_type((xu >> 16).astype(jnp.uint16), jnp.bfloat16)
    return lo, hi


_CP = pltpu.CompilerParams(dimension_semantics=("parallel",))


def _mlp_ln(x, w1, vecs, w2, T, want_packed=False):
    n, din = x.shape
    dh, do = w1.shape[1], w2.shape[1]

    def body(x_r, w1_r, w2_r, v_r, o_r, *po):
        h = jnp.maximum(_dot(x_r[...], w1_r[...]) + v_r[0:1, :dh], 0.0)
        h = _dot(h, w2_r[...]) + v_r[1:2, :do]
        y = _ln(h, v_r[2:3, :do], v_r[3:4, :do])
        o_r[...] = y
        if want_packed:
            po[0][...] = _pack_bf16(y)

    out_specs = [pl.BlockSpec((T, do), lambda i: (i, 0))]
    out_shape = [jax.ShapeDtypeStruct((n, do), F32)]
    if want_packed:
        out_specs.append(pl.BlockSpec((T, do // 2), lambda i: (i, 0)))
        out_shape.append(jax.ShapeDtypeStruct((n, do // 2), jnp.int32))
    r = pl.pallas_call(
        body,
        grid=(n // T,),
        in_specs=[
            pl.BlockSpec((T, din), lambda i: (i, 0)),
            pl.BlockSpec((din, dh), lambda i: (0, 0)),
            pl.BlockSpec((dh, do), lambda i: (0, 0)),
            pl.BlockSpec((8, vecs.shape[1]), lambda i: (0, 0)),
        ],
        out_specs=out_specs if want_packed else out_specs[0],
        out_shape=out_shape if want_packed else out_shape[0],
        compiler_params=_CP,
    )(x, w1, w2, vecs)
    return r


def _edge_mlp(xs_arr, xs_off, xd_arr, xd_off, e, w1s, w1d, w1e, vecs, w2,
              n_edges, want_enew, T_EDGE):
    nb = n_edges // T_EDGE
    has_xd = xd_arr is not None

    def body(*refs):
        if has_xd:
            xs_r, xd_r, e_r, w1s_r, w1d_r, w1e_r, w2_r, v_r, *outs = refs
        else:
            xs_r, e_r, w1s_r, w1e_r, w2_r, v_r, *outs = refs
        bf = jnp.bfloat16
        hd = D // 2
        lo, hi = _unpack_bf16(xs_r[...])
        acc = _dot(lo, w1s_r[:hd]) + _dot(hi, w1s_r[hd:])
        if has_xd:
            lo, hi = _unpack_bf16(xd_r[...])
            acc = acc + _dot(lo, w1d_r[:hd]) + _dot(hi, w1d_r[hd:])
        acc = acc + _dot(e_r[...].astype(bf), w1e_r[...]) + v_r[0:1, :]
        h = jnp.maximum(acc, 0.0)
        h = _dot(h.astype(bf), w2_r[...]) + v_r[1:2, :]
        m = _ln(h, v_r[2:3, :], v_r[3:4, :])
        outs[0][...] = m
        if want_enew:
            outs[1][...] = e_r[...] + m

    in_specs = [pl.BlockSpec((T_EDGE, D // 2), lambda i, o=xs_off: (i + o, 0))]
    inputs = [xs_arr]
    if has_xd:
        in_specs.append(
            pl.BlockSpec((T_EDGE, D // 2), lambda i, o=xd_off: (i + o, 0)))
        inputs.append(xd_arr)
    in_specs.append(pl.BlockSpec((T_EDGE, D), lambda i: (i, 0)))
    inputs.append(e)
    wspec = pl.BlockSpec((D, D), lambda i: (0, 0))
    in_specs += [wspec, wspec] if not has_xd else [wspec, wspec, wspec]
    inputs += [w1s, w1e] if not has_xd else [w1s, w1d, w1e]
    in_specs += [wspec, pl.BlockSpec((8, D), lambda i: (0, 0))]
    inputs += [w2, vecs]

    out_spec = pl.BlockSpec((T_EDGE, D), lambda i: (i, 0))
    out_shape = jax.ShapeDtypeStruct((n_edges, D), F32)
    if want_enew:
        out_specs, out_shapes = [out_spec, out_spec], [out_shape, out_shape]
    else:
        out_specs, out_shapes = out_spec, out_shape

    return pl.pallas_call(
        body,
        grid=(nb,),
        in_specs=in_specs,
        out_specs=out_specs,
        out_shape=out_shapes,
        compiler_params=_CP,
    )(*inputs)


def _node_mlp(x, aggs, w1x, w1a, vecs, w2, n_rows, T, want_packed=False):
    use_x = x is not None

    def body(*refs):
        refs = list(refs)
        p_r = refs.pop() if want_packed else None
        o_r = refs.pop()
        v_r = refs.pop()
        w2_r = refs.pop()
        x_r = refs.pop(0) if use_x else None
        agg_rs = [refs.pop(0) for _ in aggs]
        w1x_r = refs.pop(0) if use_x else None
        w1a_r = refs.pop(0) if aggs else None
        acc = v_r[0:1, :] * jnp.ones((T, 1), F32)
        if use_x:
            acc = acc + _dot(x_r[...], w1x_r[...])
        if agg_rs:
            a = agg_rs[0][...]
            for r in agg_rs[1:]:
                a = a + r[...]
            acc = acc + _dot(a, w1a_r[...])
        h = jnp.maximum(acc, 0.0)
        h = _dot(h, w2_r[...]) + v_r[1:2, :]
        m = _ln(h, v_r[2:3, :], v_r[3:4, :])
        y = (x_r[...] + m) if use_x else m
        o_r[...] = y
        if want_packed:
            p_r[...] = _pack_bf16(y)

    rspec = pl.BlockSpec((T, D), lambda i: (i, 0))
    wspec = pl.BlockSpec((D, D), lambda i: (0, 0))
    inputs, in_specs = [], []
    if use_x:
        inputs.append(x); in_specs.append(rspec)
    for a in aggs:
        inputs.append(a); in_specs.append(rspec)
    if use_x:
        inputs.append(w1x); in_specs.append(wspec)
    if aggs:
        inputs.append(w1a); in_specs.append(wspec)
    inputs += [w2, vecs]
    in_specs += [wspec, pl.BlockSpec((8, D), lambda i: (0, 0))]

    out_specs = [rspec]
    out_shape = [jax.ShapeDtypeStruct((n_rows, D), F32)]
    if want_packed:
        out_specs.append(pl.BlockSpec((T, D // 2), lambda i: (i, 0)))
        out_shape.append(jax.ShapeDtypeStruct((n_rows, D // 2), jnp.int32))
    return pl.pallas_call(
        body,
        grid=(n_rows // T,),
        in_specs=in_specs,
        out_specs=out_specs if want_packed else out_specs[0],
        out_shape=out_shape if want_packed else out_shape[0],
        compiler_params=_CP,
    )(*inputs)


def _final_mlp(x, featp, w1, vecs, w2):
    n = x.shape[0]
    dh = w1.shape[1]

    def body(x_r, f_r, w1_r, w2_r, v_r, o_r):
        h = jnp.maximum(_dot(x_r[...], w1_r[...]) + v_r[0:1, :], 0.0)
        o_r[...] = _dot(h, w2_r[...]) + v_r[1:2, :] + f_r[...]

    return pl.pallas_call(
        body,
        grid=(n // T_GRID,),
        in_specs=[
            pl.BlockSpec((T_GRID, D), lambda i: (i, 0)),
            pl.BlockSpec((T_GRID, dh), lambda i: (i, 0)),
            pl.BlockSpec((D, dh), lambda i: (0, 0)),
            pl.BlockSpec((dh, dh), lambda i: (0, 0)),
            pl.BlockSpec((8, dh), lambda i: (0, 0)),
        ],
        out_specs=pl.BlockSpec((T_GRID, dh), lambda i: (i, 0)),
        out_shape=jax.ShapeDtypeStruct((n, dh), F32),
        compiler_params=_CP,
    )(x, featp, w1, w2, vecs)


@functools.cache
def _mesh():
    return plsc.VectorSubcoreMesh(core_axis_name="c", subcore_axis_name="s",
                                  num_cores=NC, num_subcores=NS)


_CHUNK = 128


def _sc_gather(table, idx, n_out):
    return _sc_gather_raw(table, idx, n_out, table.shape[1])


def _sc_gather_raw(table, idx, n_out, w):
    bpw = n_out // NW
    nfull, rem = divmod(bpw, _CHUNK)
    cps = [_CHUNK] * nfull + ([rem] if rem else [])
    nch = len(cps)
    dt = table.dtype
    scratch = [pltpu.VMEM((bpw,), jnp.int32),
               pltpu.VMEM((_CHUNK, w), dt), pltpu.VMEM((_CHUNK, w), dt),
               pltpu.SemaphoreType.DMA, pltpu.SemaphoreType.DMA,
               pltpu.SemaphoreType.DMA, pltpu.SemaphoreType.DMA]

    @functools.partial(
        pl.kernel,
        out_type=jax.ShapeDtypeStruct((n_out, w), dt),
        mesh=_mesh(),
        scratch_types=scratch,
    )
    def k(table_hbm, idx_hbm, out_hbm, idx_v, r0, r1, sg0, sg1, sw0, sw1):
        wid = lax.axis_index("s") * NC + lax.axis_index("c")
        base = wid * bpw
        pltpu.sync_copy(idx_hbm.at[pl.ds(base, bpw)], idx_v)
        rows, sgs, sws = (r0, r1), (sg0, sg1), (sw0, sw1)

        def buf(i):
            b = i & 1
            sz = cps[i]
            return rows[b] if sz == _CHUNK else rows[b].at[pl.ds(0, sz)]

        def issue_g(i):
            sz = cps[i]
            return pltpu.async_copy(
                table_hbm.at[idx_v.at[pl.ds(i * _CHUNK, sz)]], buf(i),
                sgs[i & 1])

        def issue_w(i):
            sz = cps[i]
            return pltpu.async_copy(
                buf(i), out_hbm.at[pl.ds(base + i * _CHUNK, sz)], sws[i & 1])

        gd, wd = [None] * nch, [None] * nch
        gd[0] = issue_g(0)
        for i in range(nch):
            if i + 1 < nch:
                if i >= 1:
                    wd[i - 1].wait()
                gd[i + 1] = issue_g(i + 1)
            gd[i].wait()
            wd[i] = issue_w(i)
        wd[nch - 1].wait()
        if nch >= 2:
            wd[nch - 2].wait()

    return k(table, idx)


def _sc_scatter(m, dst, zeros, n_edges, n_nodes):
    dh = D // NC
    epw = n_edges // NS
    nch = epw // _CHUNK
    assert nch * _CHUNK == epw
    rps = n_nodes // NS
    scratch = [pltpu.VMEM((1, _CHUNK), jnp.int32), pltpu.VMEM((1, _CHUNK), jnp.int32),
               pltpu.VMEM((_CHUNK, dh), F32), pltpu.VMEM((_CHUNK, dh), F32),
               pltpu.SemaphoreType.DMA, pltpu.SemaphoreType.DMA,
               pltpu.SemaphoreType.DMA, pltpu.SemaphoreType.DMA,
               pltpu.SemaphoreType.DMA, pltpu.SemaphoreType.DMA,
               pltpu.VMEM_SHARED((n_nodes, dh), F32)]

    @functools.partial(
        pl.kernel,
        out_type=jax.ShapeDtypeStruct((n_nodes, D), F32),
        mesh=_mesh(),
        scratch_types=scratch,
    )
    def k(m_hbm, dst_hbm, z_hbm, out_hbm, i0, i1, m0, m1,
          si0, si1, sm0, sm1, ss0, ss1, acc):
        cid = lax.axis_index("c")
        sid = lax.axis_index("s")
        col = cid * dh
        pltpu.sync_copy(z_hbm, acc.at[pl.ds(sid * rps, rps)])
        plsc.subcore_barrier()
        base = sid * epw
        idxs, mbufs = (i0, i1), (m0, m1)
        sis, sms, sss = (si0, si1), (sm0, sm1), (ss0, ss1)

        def issue_l(i):
            b = i & 1
            off = base + i * _CHUNK
            di = pltpu.async_copy(dst_hbm.at[pl.ds(off, _CHUNK)],
                                  idxs[b].at[0], sis[b])
            dm = pltpu.async_copy(m_hbm.at[pl.ds(off, _CHUNK), pl.ds(col, dh)],
                                  mbufs[b], sms[b])
            return di, dm

        def issue_s(i):
            b = i & 1
            return pltpu.async_copy(mbufs[b], acc.at[idxs[b].at[0]], sss[b],
                                    add=True)

        ld, sd = [None] * nch, [None] * nch
        ld[0] = issue_l(0)
        for i in range(nch):
            if i + 1 < nch:
                if i >= 1:
                    sd[i - 1].wait()
                ld[i + 1] = issue_l(i + 1)
            ld[i][0].wait()
            ld[i][1].wait()
            sd[i] = issue_s(i)
        sd[nch - 1].wait()
        if nch >= 2:
            sd[nch - 2].wait()
        plsc.subcore_barrier()
        pltpu.sync_copy(acc.at[pl.ds(sid * rps, rps)],
                        out_hbm.at[pl.ds(sid * rps, rps), pl.ds(col, dh)])

    return k(m, dst, zeros)


def _pad_idx(a, n, fill_start, fill_mod):
    npad = n - a.shape[0]
    pad = fill_start + (jnp.arange(npad, dtype=jnp.int32) % fill_mod)
    return jnp.concatenate([a.astype(jnp.int32), pad])


def _pad2(a, rows, cols):
    return jnp.pad(a, ((0, rows - a.shape[0]), (0, cols - a.shape[1])))


def kernel(features, params, graph):
    f = features[0]
    P, G = params, graph

    fpad = _pad2(f, N_GRID, 128)
    zeros_lat = jnp.zeros((LAT_P // NS, D // NC), F32)
    zeros_grid = jnp.zeros((GRID_P // NS, D // NC), F32)

    enc_src = _pad_idx(G["enc_src"], EP_E, 0, N_GRID)
    enc_dst = _pad_idx(G["enc_dst"], EP_E, N_LAT, LAT_P - N_LAT)
    proc_srcdst = jnp.concatenate([
        _pad_idx(G["proc_src"], EP_P, 0, N_LAT),
        _pad_idx(G["proc_dst"], EP_P, 0, N_LAT)])
    proc_dst = _pad_idx(G["proc_dst"], EP_P, N_LAT, LAT_P - N_LAT)
    dec_srcdst = jnp.concatenate([
        _pad_idx(G["dec_src"], EP_E, 0, N_LAT),
        LAT_P + _pad_idx(G["dec_dst"], EP_E, 0, N_GRID)])
    dec_dst_s = _pad_idx(G["dec_dst"], EP_E, N_GRID, GRID_P - N_GRID)

    pne = P["node_encoder"]
    x_grid, x_grid_pk = _mlp_ln(fpad, _pad2(pne["w1"], 128, D),
                                _vecs(pne["b1"], pne["b2"], pne["ln_g"],
                                      pne["ln_b"]),
                                pne["w2"], T_GRID, want_packed=True)

    def enc_attr(p, attr, n_pad):
        return _mlp_ln(_pad2(attr, n_pad, 128), _pad2(p["w1"], 128, D),
                       _vecs(p["b1"], p["b2"], p["ln_g"], p["ln_b"]),
                       p["w2"], T_EE)

    e_enc = enc_attr(P["enc_edge_encoder"], G["enc_attr"], EP_E)
    ep = enc_attr(P["proc_edge_encoder"], G["proc_attr"], EP_P)
    ed = enc_attr(P["dec_edge_encoder"], G["dec_attr"], EP_E)

    def edge_w(p):
        w1 = p["w1"].astype(jnp.bfloat16)
        return (w1[:D], w1[D:2 * D], w1[2 * D:],
                _vecs(p["b1"], p["b2"], p["ln_g"], p["ln_b"]),
                p["w2"].astype(jnp.bfloat16))

    def node_w(p):
        w1 = p["w1"]
        return (w1[:D], w1[D:],
                _vecs(p["b1"], p["b2"], p["ln_g"], p["ln_b"]), p["w2"])

    w1s, _, w1e, vecs, w2 = edge_w(P["enc_block"]["edge"])
    xs = _sc_gather(x_grid_pk, enc_src, EP_E)
    m = _edge_mlp(xs, 0, None, 0, e_enc, w1s, None, w1e, vecs, w2, EP_E, False, T_EE)
    agg = _sc_scatter(m, enc_dst, zeros_lat, EP_E, LAT_P)
    w1x, w1a, nvecs, nw2 = node_w(P["enc_block"]["node"])
    x_lat, x_lat_pk = _node_mlp(None, [agg], None, w1a, nvecs, nw2, LAT_P,
                                T_LAT, want_packed=True)
    x_grid, x_grid_pk = _node_mlp(x_grid, [], w1x, None, nvecs, nw2, N_GRID,
                                  T_GRID, want_packed=True)

    nb_p = EP_P // T_EP
    for bp in P["proc_blocks"]:
        w1s, w1d, w1e, vecs, w2 = edge_w(bp["edge"])
        rows = _sc_gather(x_lat_pk, proc_srcdst, 2 * EP_P)
        m, ep = _edge_mlp(rows, 0, rows, nb_p, ep, w1s, w1d, w1e, vecs, w2,
                          EP_P, True, T_EP)
        agg = _sc_scatter(m, proc_dst, zeros_lat, EP_P, LAT_P)
        w1x, w1a, nvecs, nw2 = node_w(bp["node"])
        x_lat, x_lat_pk = _node_mlp(x_lat, [agg], w1x, w1a, nvecs, nw2, LAT_P,
                                    T_LAT, want_packed=True)

    w1s, w1d, w1e, vecs, w2 = edge_w(P["dec_block"]["edge"])
    table = jnp.concatenate([x_lat_pk, x_grid_pk])
    rows = _sc_gather(table, dec_srcdst, 2 * EP_E)
    nb_e = EP_E // T_EE
    m = _edge_mlp(rows, 0, rows, nb_e, ed, w1s, w1d, w1e, vecs, w2, EP_E, False,
                  T_EE)
    agg = _sc_scatter(m, dec_dst_s, zeros_grid, EP_E, GRID_P)
    w1x, w1a, nvecs, nw2 = node_w(P["dec_block"]["node"])
    x_grid = _node_mlp(x_grid, [agg], w1x, w1a, nvecs, nw2, N_GRID, T_GRID)

    pd = P["node_decoder"]
    out = _final_mlp(x_grid, fpad, pd["w1"],
                     _vecs(pd["b1"], jnp.pad(pd["b2"], (0, 128 - D_IN))),
                     _pad2(pd["w2"], 128, 128))
    return out[:, :D_IN][None]
